# Initial kernel scaffold; baseline (speedup 1.0000x reference)
import functools

import jax
import jax.numpy as jnp
from jax import lax
from jax.experimental import pallas as pl
from jax.experimental.pallas import tpu as pltpu
from jax.experimental.pallas import tpu_sc as plsc

N = 10000
R = 10240
E = 320000
NC = 2
NS = 16
NW = NC * NS
CH = 128
KC = (E + NW * CH - 1) // (NW * CH)
EP = NW * CH * KC
EPW = CH * KC
PAD_ROW = N
RPW = R // NS
DKC = RPW // CH

_mesh = plsc.VectorSubcoreMesh(core_axis_name="c", subcore_axis_name="s")


def _fill_rows(ref, nrows, ncols, value):
    vec = jnp.full((16,), value, jnp.float32)

    def body(i, carry):
        for k in range(ncols // 16):
            ref[i, pl.ds(16 * k, 16)] = vec
        return carry

    lax.fori_loop(0, nrows, body, 0)


@functools.partial(
    pl.kernel,
    out_type=(
        jax.ShapeDtypeStruct((NC, R, 16), jnp.float32),
        jax.ShapeDtypeStruct((NC, R, 16), jnp.float32),
    ),
    mesh=_mesh,
    scratch_types=[
        pltpu.VMEM((KC, CH), jnp.int32),
        pltpu.VMEM((KC, CH), jnp.int32),
        pltpu.VMEM((CH, 16), jnp.float32),
        pltpu.VMEM_SHARED((R, 16), jnp.float32),
        pltpu.VMEM_SHARED((R, 16), jnp.float32),
    ],
)
def _sc_degrees(src_hbm, dst_hbm, out_s_hbm, out_d_hbm,
                src_v, dst_v, stage_v, acc_s, acc_d):
    cid = lax.axis_index("c")
    sid = lax.axis_index("s")
    wid = cid * NS + sid

    _fill_rows(stage_v, CH, 16, 0.0)
    for k in range(DKC):
        r0 = sid * RPW + k * CH
        pltpu.sync_copy(stage_v, acc_s.at[pl.ds(r0, CH)])
        pltpu.sync_copy(stage_v, acc_d.at[pl.ds(r0, CH)])
    _fill_rows(stage_v, CH, 16, 1.0)
    pltpu.sync_copy(src_hbm.at[wid], src_v)
    pltpu.sync_copy(dst_hbm.at[wid], dst_v)
    plsc.subcore_barrier()

    def body(j, carry):
        pltpu.sync_copy(stage_v, acc_s.at[src_v.at[j]], add=True)
        pltpu.sync_copy(stage_v, acc_d.at[dst_v.at[j]], add=True)
        return carry

    lax.fori_loop(0, KC, body, 0)
    plsc.subcore_barrier()

    for k in range(DKC):
        r0 = sid * RPW + k * CH
        pltpu.sync_copy(acc_s.at[pl.ds(r0, CH)], stage_v)
        pltpu.sync_copy(stage_v, out_s_hbm.at[cid, pl.ds(r0, CH)])
        pltpu.sync_copy(acc_d.at[pl.ds(r0, CH)], stage_v)
        pltpu.sync_copy(stage_v, out_d_hbm.at[cid, pl.ds(r0, CH)])


def _make_sc_agg(D):
    @functools.partial(
        pl.kernel,
        out_type=jax.ShapeDtypeStruct((NC, R, D), jnp.float32),
        mesh=_mesh,
        scratch_types=[
            pltpu.VMEM((KC, CH), jnp.int32),
            pltpu.VMEM((KC, CH), jnp.int32),
            pltpu.VMEM((CH, D), jnp.float32),
            pltpu.VMEM_SHARED((R, D), jnp.float32),
            pltpu.SemaphoreType.DMA,
        ],
    )
    def agg(table_hbm, src_hbm, dst_hbm, out_hbm, src_v, dst_v, rows_v, acc, sem):
        cid = lax.axis_index("c")
        sid = lax.axis_index("s")
        wid = cid * NS + sid

        _fill_rows(rows_v, CH, D, 0.0)
        for k in range(DKC):
            r0 = sid * RPW + k * CH
            pltpu.sync_copy(rows_v, acc.at[pl.ds(r0, CH)])
        pltpu.sync_copy(src_hbm.at[wid], src_v)
        pltpu.sync_copy(dst_hbm.at[wid], dst_v)
        plsc.subcore_barrier()

        def body(j, carry):
            pltpu.async_copy(table_hbm.at[src_v.at[j]], rows_v, sem).wait()
            pltpu.sync_copy(rows_v, acc.at[dst_v.at[j]], add=True)
            return carry

        lax.fori_loop(0, KC, body, 0)
        plsc.subcore_barrier()

        for k in range(DKC):
            r0 = sid * RPW + k * CH
            pltpu.sync_copy(acc.at[pl.ds(r0, CH)], rows_v)
            pltpu.sync_copy(rows_v, out_hbm.at[cid, pl.ds(r0, CH)])

    return agg


_sc_agg128 = _make_sc_agg(128)
_sc_agg16 = _make_sc_agg(16)


BLK = 512


def _norm_from(deg_ref):
    deg = deg_ref[0, :, 0] + deg_ref[1, :, 0]
    return jnp.where(deg > 0, lax.rsqrt(jnp.maximum(deg, 1.0)), 0.0)


def _tc_h1p_body(x_ref, w_ref, degs_ref, o_ref):
    h = jnp.dot(x_ref[...], w_ref[...], preferred_element_type=jnp.float32)
    o_ref[...] = h * _norm_from(degs_ref)[:, None]


def _tc_h1p(x_pad, W1, deg_s):
    return pl.pallas_call(
        _tc_h1p_body,
        grid=(R // BLK,),
        in_specs=[
            pl.BlockSpec((BLK, 128), lambda i: (i, 0)),
            pl.BlockSpec((128, 128), lambda i: (0, 0)),
            pl.BlockSpec((NC, BLK, 16), lambda i: (0, i, 0)),
        ],
        out_specs=pl.BlockSpec((BLK, 128), lambda i: (i, 0)),
        out_shape=jax.ShapeDtypeStruct((R, 128), jnp.float32),
    )(x_pad, W1, deg_s)


def _tc_mid_body(p_ref, degs_ref, degd_ref, b1_ref, w2_ref, o_ref):
    agg = p_ref[0] + p_ref[1]
    h = jax.nn.relu(agg * _norm_from(degd_ref)[:, None] + b1_ref[...])
    h2 = jnp.dot(h, w2_ref[...], preferred_element_type=jnp.float32)
    o_ref[...] = h2 * _norm_from(degs_ref)[:, None]


def _tc_mid(parts1, deg_s, deg_d, b1, W2):
    return pl.pallas_call(
        _tc_mid_body,
        grid=(R // BLK,),
        in_specs=[
            pl.BlockSpec((NC, BLK, 128), lambda i: (0, i, 0)),
            pl.BlockSpec((NC, BLK, 16), lambda i: (0, i, 0)),
            pl.BlockSpec((NC, BLK, 16), lambda i: (0, i, 0)),
            pl.BlockSpec((1, 128), lambda i: (0, 0)),
            pl.BlockSpec((128, 16), lambda i: (0, 0)),
        ],
        out_specs=pl.BlockSpec((BLK, 16), lambda i: (i, 0)),
        out_shape=jax.ShapeDtypeStruct((R, 16), jnp.float32),
    )(parts1, deg_s, deg_d, b1, W2)


def _tc_softmax_body(p_ref, degd_ref, b2_ref, o_ref):
    agg = p_ref[0] + p_ref[1]
    z = agg * _norm_from(degd_ref)[:, None] + b2_ref[...]
    z = z - jnp.max(z, axis=1, keepdims=True)
    ez = jnp.exp(z)
    o_ref[...] = ez / jnp.sum(ez, axis=1, keepdims=True)


def _tc_softmax(parts2, deg_d, b2):
    return pl.pallas_call(
        _tc_softmax_body,
        grid=(R // BLK,),
        in_specs=[
            pl.BlockSpec((NC, BLK, 16), lambda i: (0, i, 0)),
            pl.BlockSpec((NC, BLK, 16), lambda i: (0, i, 0)),
            pl.BlockSpec((1, 16), lambda i: (0, 0)),
        ],
        out_specs=pl.BlockSpec((BLK, 16), lambda i: (i, 0)),
        out_shape=jax.ShapeDtypeStruct((R, 16), jnp.float32),
    )(parts2, deg_d, b2)


def kernel(edge_index, inputs, W1, b1, W2, b2):
    src = edge_index[0].astype(jnp.int32)
    dst = edge_index[1].astype(jnp.int32)
    pad = jnp.full((EP - E,), PAD_ROW, jnp.int32)
    src3 = jnp.concatenate([src, pad]).reshape(NW, KC, CH)
    dst3 = jnp.concatenate([dst, pad]).reshape(NW, KC, CH)

    deg_s, deg_d = _sc_degrees(src3, dst3)

    x_pad = jnp.pad(inputs, ((0, R - N), (0, 0)))
    h1p = _tc_h1p(x_pad, W1, deg_s)
    parts1 = _sc_agg128(h1p, src3, dst3)
    h2p = _tc_mid(parts1, deg_s, deg_d, b1.reshape(1, 128), W2)
    parts2 = _sc_agg16(h2p, src3, dst3)
    out = _tc_softmax(parts2, deg_d, b2.reshape(1, 16))
    return out[:N]


# speedup vs baseline: 11.4063x; 11.4063x over previous
import functools

import jax
import jax.numpy as jnp
from jax import lax
from jax.experimental import pallas as pl
from jax.experimental.pallas import tpu as pltpu
from jax.experimental.pallas import tpu_sc as plsc

N = 10000
R = 10240
E = 320000
NC = 2
NS = 16
NW = NC * NS
CH = 128
KC = (E + NW * CH - 1) // (NW * CH)
EP = NW * CH * KC
EPW = CH * KC
PAD_ROW = N
RPW = R // NS
DKC = RPW // CH

_mesh = plsc.VectorSubcoreMesh(core_axis_name="c", subcore_axis_name="s")
_sc_params = pltpu.CompilerParams(use_tc_tiling_on_sc=False)


def _fill_rows(ref, nrows, ncols, value):
    vec = jnp.full((16,), value, jnp.float32)

    def body(i, carry):
        for k in range(ncols // 16):
            ref[i, pl.ds(16 * k, 16)] = vec
        return carry

    lax.fori_loop(0, nrows, body, 0)


@functools.partial(
    pl.kernel,
    out_type=(
        jax.ShapeDtypeStruct((NC, R, 16), jnp.float32),
        jax.ShapeDtypeStruct((NC, R, 16), jnp.float32),
    ),
    mesh=_mesh,
    compiler_params=_sc_params,
    scratch_types=[
        pltpu.VMEM((KC, CH), jnp.int32),
        pltpu.VMEM((KC, CH), jnp.int32),
        pltpu.VMEM((CH, 16), jnp.float32),
        pltpu.VMEM_SHARED((R, 16), jnp.float32),
        pltpu.VMEM_SHARED((R, 16), jnp.float32),
    ],
)
def _sc_degrees(src_hbm, dst_hbm, out_s_hbm, out_d_hbm,
                src_v, dst_v, stage_v, acc_s, acc_d):
    cid = lax.axis_index("c")
    sid = lax.axis_index("s")
    wid = cid * NS + sid

    _fill_rows(stage_v, CH, 16, 0.0)
    for k in range(DKC):
        r0 = sid * RPW + k * CH
        pltpu.sync_copy(stage_v, acc_s.at[pl.ds(r0, CH)])
        pltpu.sync_copy(stage_v, acc_d.at[pl.ds(r0, CH)])
    _fill_rows(stage_v, CH, 16, 1.0)
    pltpu.sync_copy(src_hbm.at[wid], src_v)
    pltpu.sync_copy(dst_hbm.at[wid], dst_v)
    plsc.subcore_barrier()

    def body(j, carry):
        pltpu.sync_copy(stage_v, acc_s.at[src_v.at[j]], add=True)
        pltpu.sync_copy(stage_v, acc_d.at[dst_v.at[j]], add=True)
        return carry

    lax.fori_loop(0, KC, body, 0)
    plsc.subcore_barrier()

    for k in range(DKC):
        r0 = sid * RPW + k * CH
        pltpu.sync_copy(acc_s.at[pl.ds(r0, CH)], stage_v)
        pltpu.sync_copy(stage_v, out_s_hbm.at[cid, pl.ds(r0, CH)])
        pltpu.sync_copy(acc_d.at[pl.ds(r0, CH)], stage_v)
        pltpu.sync_copy(stage_v, out_d_hbm.at[cid, pl.ds(r0, CH)])


def _make_sc_agg(D):
    @functools.partial(
        pl.kernel,
        out_type=jax.ShapeDtypeStruct((NC, R, D), jnp.float32),
        mesh=_mesh,
        compiler_params=_sc_params,
        scratch_types=[
            pltpu.VMEM((KC, CH), jnp.int32),
            pltpu.VMEM((KC, CH), jnp.int32),
            pltpu.VMEM((CH, D), jnp.float32),
            pltpu.VMEM_SHARED((R, D), jnp.float32),
            pltpu.SemaphoreType.DMA,
        ],
    )
    def agg(table_hbm, src_hbm, dst_hbm, out_hbm, src_v, dst_v, rows_v, acc, sem):
        cid = lax.axis_index("c")
        sid = lax.axis_index("s")
        wid = cid * NS + sid

        _fill_rows(rows_v, CH, D, 0.0)
        for k in range(DKC):
            r0 = sid * RPW + k * CH
            pltpu.sync_copy(rows_v, acc.at[pl.ds(r0, CH)])
        pltpu.sync_copy(src_hbm.at[wid], src_v)
        pltpu.sync_copy(dst_hbm.at[wid], dst_v)
        plsc.subcore_barrier()

        def body(j, carry):
            pltpu.async_copy(table_hbm.at[src_v.at[j]], rows_v, sem).wait()
            pltpu.sync_copy(rows_v, acc.at[dst_v.at[j]], add=True)
            return carry

        lax.fori_loop(0, KC, body, 0)
        plsc.subcore_barrier()

        for k in range(DKC):
            r0 = sid * RPW + k * CH
            pltpu.sync_copy(acc.at[pl.ds(r0, CH)], rows_v)
            pltpu.sync_copy(rows_v, out_hbm.at[cid, pl.ds(r0, CH)])

    return agg


_sc_agg128 = _make_sc_agg(128)
_sc_agg16 = _make_sc_agg(16)


BLK = 512


def _norm_from(deg_ref):
    deg = deg_ref[0, :, 0] + deg_ref[1, :, 0]
    return jnp.where(deg > 0, lax.rsqrt(jnp.maximum(deg, 1.0)), 0.0)


def _tc_h1p_body(x_ref, w_ref, degs_ref, o_ref):
    h = jnp.dot(x_ref[...], w_ref[...], preferred_element_type=jnp.float32)
    o_ref[...] = h * _norm_from(degs_ref)[:, None]


def _tc_h1p(x_pad, W1, deg_s):
    return pl.pallas_call(
        _tc_h1p_body,
        grid=(R // BLK,),
        in_specs=[
            pl.BlockSpec((BLK, 128), lambda i: (i, 0)),
            pl.BlockSpec((128, 128), lambda i: (0, 0)),
            pl.BlockSpec((NC, BLK, 16), lambda i: (0, i, 0)),
        ],
        out_specs=pl.BlockSpec((BLK, 128), lambda i: (i, 0)),
        out_shape=jax.ShapeDtypeStruct((R, 128), jnp.float32),
    )(x_pad, W1, deg_s)


def _tc_mid_body(p_ref, degs_ref, degd_ref, b1_ref, w2_ref, o_ref):
    agg = p_ref[0] + p_ref[1]
    h = jax.nn.relu(agg * _norm_from(degd_ref)[:, None] + b1_ref[...])
    h2 = jnp.dot(h, w2_ref[...], preferred_element_type=jnp.float32)
    o_ref[...] = h2 * _norm_from(degs_ref)[:, None]


def _tc_mid(parts1, deg_s, deg_d, b1, W2):
    return pl.pallas_call(
        _tc_mid_body,
        grid=(R // BLK,),
        in_specs=[
            pl.BlockSpec((NC, BLK, 128), lambda i: (0, i, 0)),
            pl.BlockSpec((NC, BLK, 16), lambda i: (0, i, 0)),
            pl.BlockSpec((NC, BLK, 16), lambda i: (0, i, 0)),
            pl.BlockSpec((1, 128), lambda i: (0, 0)),
            pl.BlockSpec((128, 16), lambda i: (0, 0)),
        ],
        out_specs=pl.BlockSpec((BLK, 16), lambda i: (i, 0)),
        out_shape=jax.ShapeDtypeStruct((R, 16), jnp.float32),
    )(parts1, deg_s, deg_d, b1, W2)


def _tc_softmax_body(p_ref, degd_ref, b2_ref, o_ref):
    agg = p_ref[0] + p_ref[1]
    z = agg * _norm_from(degd_ref)[:, None] + b2_ref[...]
    z = z - jnp.max(z, axis=1, keepdims=True)
    ez = jnp.exp(z)
    o_ref[...] = ez / jnp.sum(ez, axis=1, keepdims=True)


def _tc_softmax(parts2, deg_d, b2):
    return pl.pallas_call(
        _tc_softmax_body,
        grid=(R // BLK,),
        in_specs=[
            pl.BlockSpec((NC, BLK, 16), lambda i: (0, i, 0)),
            pl.BlockSpec((NC, BLK, 16), lambda i: (0, i, 0)),
            pl.BlockSpec((1, 16), lambda i: (0, 0)),
        ],
        out_specs=pl.BlockSpec((BLK, 16), lambda i: (i, 0)),
        out_shape=jax.ShapeDtypeStruct((R, 16), jnp.float32),
    )(parts2, deg_d, b2)


def kernel(edge_index, inputs, W1, b1, W2, b2):
    src = edge_index[0].astype(jnp.int32)
    dst = edge_index[1].astype(jnp.int32)
    pad = jnp.full((EP - E,), PAD_ROW, jnp.int32)
    src3 = jnp.concatenate([src, pad]).reshape(NW, KC, CH)
    dst3 = jnp.concatenate([dst, pad]).reshape(NW, KC, CH)

    deg_s, deg_d = _sc_degrees(src3, dst3)

    x_pad = jnp.pad(inputs, ((0, R - N), (0, 0)))
    h1p = _tc_h1p(x_pad, W1, deg_s)
    parts1 = _sc_agg128(h1p, src3, dst3)
    h2p = _tc_mid(parts1, deg_s, deg_d, b1.reshape(1, 128), W2)
    parts2 = _sc_agg16(h2p, src3, dst3)
    out = _tc_softmax(parts2, deg_d, b2.reshape(1, 16))
    return out[:N]


# speedup vs baseline: 16.7313x; 1.4669x over previous
import functools

import jax
import jax.numpy as jnp
from jax import lax
from jax.experimental import pallas as pl
from jax.experimental.pallas import tpu as pltpu
from jax.experimental.pallas import tpu_sc as plsc

N = 10000
R = 10112
E = 320000
NC = 2
NS = 16
NW = NC * NS
CH = 112
KC = (E + NW * CH - 1) // (NW * CH)
EP = NW * CH * KC
EPW = CH * KC
PAD_ROW = N
RPW = R // NS
DCH = 79
DKC = RPW // DCH

_mesh = plsc.VectorSubcoreMesh(core_axis_name="c", subcore_axis_name="s")
_sc_params = pltpu.CompilerParams(use_tc_tiling_on_sc=False)


def _fill_rows(ref, nrows, ncols, value):
    vec = jnp.full((16,), value, jnp.float32)

    def body(i, carry):
        for k in range(ncols // 16):
            ref[i, pl.ds(16 * k, 16)] = vec
        return carry

    lax.fori_loop(0, nrows, body, 0)


@functools.partial(
    pl.kernel,
    out_type=(
        jax.ShapeDtypeStruct((NC, R, 16), jnp.float32),
        jax.ShapeDtypeStruct((NC, R, 16), jnp.float32),
    ),
    mesh=_mesh,
    compiler_params=_sc_params,
    scratch_types=[
        pltpu.VMEM((KC, CH), jnp.int32),
        pltpu.VMEM((KC, CH), jnp.int32),
        pltpu.VMEM((CH, 16), jnp.float32),
        pltpu.VMEM_SHARED((R, 16), jnp.float32),
        pltpu.VMEM_SHARED((R, 16), jnp.float32),
        pltpu.SemaphoreType.DMA,
        pltpu.SemaphoreType.DMA,
    ],
)
def _sc_degrees(src_hbm, dst_hbm, out_s_hbm, out_d_hbm,
                src_v, dst_v, stage_v, acc_s, acc_d, sem_s, sem_d):
    cid = lax.axis_index("c")
    sid = lax.axis_index("s")
    wid = cid * NS + sid

    _fill_rows(stage_v, CH, 16, 0.0)
    for k in range(DKC):
        r0 = sid * RPW + k * DCH
        pltpu.sync_copy(stage_v.at[pl.ds(0, DCH)], acc_s.at[pl.ds(r0, DCH)])
        pltpu.sync_copy(stage_v.at[pl.ds(0, DCH)], acc_d.at[pl.ds(r0, DCH)])
    _fill_rows(stage_v, CH, 16, 1.0)
    pltpu.sync_copy(src_hbm.at[wid], src_v)
    pltpu.sync_copy(dst_hbm.at[wid], dst_v)
    plsc.subcore_barrier()

    descs = [None] * KC
    for j in range(KC):
        descs[j] = (
            pltpu.async_copy(stage_v, acc_s.at[src_v.at[j]], sem_s, add=True),
            pltpu.async_copy(stage_v, acc_d.at[dst_v.at[j]], sem_d, add=True),
        )
        if j >= 2:
            descs[j - 2][0].wait()
            descs[j - 2][1].wait()
    for j in range(max(KC - 2, 0), KC):
        descs[j][0].wait()
        descs[j][1].wait()
    plsc.subcore_barrier()

    for k in range(DKC):
        r0 = sid * RPW + k * DCH
        pltpu.sync_copy(acc_s.at[pl.ds(r0, DCH)], stage_v.at[pl.ds(0, DCH)])
        pltpu.sync_copy(stage_v.at[pl.ds(0, DCH)], out_s_hbm.at[cid, pl.ds(r0, DCH)])
        pltpu.sync_copy(acc_d.at[pl.ds(r0, DCH)], stage_v.at[pl.ds(0, DCH)])
        pltpu.sync_copy(stage_v.at[pl.ds(0, DCH)], out_d_hbm.at[cid, pl.ds(r0, DCH)])


def _make_sc_agg(D):
    @functools.partial(
        pl.kernel,
        out_type=jax.ShapeDtypeStruct((NC, R, D), jnp.float32),
        mesh=_mesh,
        compiler_params=_sc_params,
        scratch_types=[
            pltpu.VMEM((KC, CH), jnp.int32),
            pltpu.VMEM((KC, CH), jnp.int32),
            pltpu.VMEM((CH, D), jnp.float32),
            pltpu.VMEM((CH, D), jnp.float32),
            pltpu.VMEM_SHARED((R, D), jnp.float32),
            pltpu.SemaphoreType.DMA,
            pltpu.SemaphoreType.DMA,
        ],
    )
    def agg(table_hbm, src_hbm, dst_hbm, out_hbm,
            src_v, dst_v, buf0, buf1, acc, sem0, sem1):
        cid = lax.axis_index("c")
        sid = lax.axis_index("s")
        wid = cid * NS + sid
        bufs = (buf0, buf1)
        sems = (sem0, sem1)

        _fill_rows(buf0, CH, D, 0.0)
        for k in range(DKC):
            r0 = sid * RPW + k * DCH
            pltpu.sync_copy(buf0.at[pl.ds(0, DCH)], acc.at[pl.ds(r0, DCH)])
        pltpu.sync_copy(src_hbm.at[wid], src_v)
        pltpu.sync_copy(dst_hbm.at[wid], dst_v)
        plsc.subcore_barrier()

        desc = [None] * KC
        desc[0] = pltpu.async_copy(table_hbm.at[src_v.at[0]], buf0, sem0)
        for j in range(KC):
            if j + 1 < KC:
                desc[j + 1] = pltpu.async_copy(
                    table_hbm.at[src_v.at[j + 1]], bufs[(j + 1) % 2],
                    sems[(j + 1) % 2])
            desc[j].wait()
            pltpu.sync_copy(bufs[j % 2], acc.at[dst_v.at[j]], add=True)
        plsc.subcore_barrier()

        for k in range(DKC):
            r0 = sid * RPW + k * DCH
            pltpu.sync_copy(acc.at[pl.ds(r0, DCH)], buf0.at[pl.ds(0, DCH)])
            pltpu.sync_copy(buf0.at[pl.ds(0, DCH)], out_hbm.at[cid, pl.ds(r0, DCH)])

    return agg


_sc_agg128 = _make_sc_agg(128)
_sc_agg16 = _make_sc_agg(16)


BLK = 632


def _norm_from(deg_ref):
    deg = deg_ref[0, :, 0] + deg_ref[1, :, 0]
    return jnp.where(deg > 0, lax.rsqrt(jnp.maximum(deg, 1.0)), 0.0)


def _tc_h1p_body(x_ref, w_ref, degs_ref, o_ref):
    h = jnp.dot(x_ref[...], w_ref[...], preferred_element_type=jnp.float32)
    o_ref[...] = h * _norm_from(degs_ref)[:, None]


def _tc_h1p(x_pad, W1, deg_s):
    return pl.pallas_call(
        _tc_h1p_body,
        grid=(R // BLK,),
        in_specs=[
            pl.BlockSpec((BLK, 128), lambda i: (i, 0)),
            pl.BlockSpec((128, 128), lambda i: (0, 0)),
            pl.BlockSpec((NC, BLK, 16), lambda i: (0, i, 0)),
        ],
        out_specs=pl.BlockSpec((BLK, 128), lambda i: (i, 0)),
        out_shape=jax.ShapeDtypeStruct((R, 128), jnp.float32),
    )(x_pad, W1, deg_s)


def _tc_mid_body(p_ref, degs_ref, degd_ref, b1_ref, w2_ref, o_ref):
    agg = p_ref[0] + p_ref[1]
    h = jax.nn.relu(agg * _norm_from(degd_ref)[:, None] + b1_ref[...])
    h2 = jnp.dot(h, w2_ref[...], preferred_element_type=jnp.float32)
    o_ref[...] = h2 * _norm_from(degs_ref)[:, None]


def _tc_mid(parts1, deg_s, deg_d, b1, W2):
    return pl.pallas_call(
        _tc_mid_body,
        grid=(R // BLK,),
        in_specs=[
            pl.BlockSpec((NC, BLK, 128), lambda i: (0, i, 0)),
            pl.BlockSpec((NC, BLK, 16), lambda i: (0, i, 0)),
            pl.BlockSpec((NC, BLK, 16), lambda i: (0, i, 0)),
            pl.BlockSpec((1, 128), lambda i: (0, 0)),
            pl.BlockSpec((128, 16), lambda i: (0, 0)),
        ],
        out_specs=pl.BlockSpec((BLK, 16), lambda i: (i, 0)),
        out_shape=jax.ShapeDtypeStruct((R, 16), jnp.float32),
    )(parts1, deg_s, deg_d, b1, W2)


def _tc_softmax_body(p_ref, degd_ref, b2_ref, o_ref):
    agg = p_ref[0] + p_ref[1]
    z = agg * _norm_from(degd_ref)[:, None] + b2_ref[...]
    z = z - jnp.max(z, axis=1, keepdims=True)
    ez = jnp.exp(z)
    o_ref[...] = ez / jnp.sum(ez, axis=1, keepdims=True)


def _tc_softmax(parts2, deg_d, b2):
    return pl.pallas_call(
        _tc_softmax_body,
        grid=(R // BLK,),
        in_specs=[
            pl.BlockSpec((NC, BLK, 16), lambda i: (0, i, 0)),
            pl.BlockSpec((NC, BLK, 16), lambda i: (0, i, 0)),
            pl.BlockSpec((1, 16), lambda i: (0, 0)),
        ],
        out_specs=pl.BlockSpec((BLK, 16), lambda i: (i, 0)),
        out_shape=jax.ShapeDtypeStruct((R, 16), jnp.float32),
    )(parts2, deg_d, b2)


def kernel(edge_index, inputs, W1, b1, W2, b2):
    src = edge_index[0].astype(jnp.int32)
    dst = edge_index[1].astype(jnp.int32)
    pad = jnp.full((EP - E,), PAD_ROW, jnp.int32)
    src3 = jnp.concatenate([src, pad]).reshape(NW, KC, CH)
    dst3 = jnp.concatenate([dst, pad]).reshape(NW, KC, CH)

    deg_s, deg_d = _sc_degrees(src3, dst3)

    x_pad = jnp.pad(inputs, ((0, R - N), (0, 0)))
    h1p = _tc_h1p(x_pad, W1, deg_s)
    parts1 = _sc_agg128(h1p, src3, dst3)
    h2p = _tc_mid(parts1, deg_s, deg_d, b1.reshape(1, 128), W2)
    parts2 = _sc_agg16(h2p, src3, dst3)
    out = _tc_softmax(parts2, deg_d, b2.reshape(1, 16))
    return out[:N]


# speedup vs baseline: 22.2071x; 1.3273x over previous
import functools

import jax
import jax.numpy as jnp
from jax import lax
from jax.experimental import pallas as pl
from jax.experimental.pallas import tpu as pltpu
from jax.experimental.pallas import tpu_sc as plsc

N = 10000
R = 10112
E = 320000
NC = 2
NS = 16
NW = NC * NS
CH = 112
KC = (E + NW * CH - 1) // (NW * CH)
EP = NW * CH * KC
EPW = CH * KC
PAD_ROW = N
RPW = R // NS
DCH = 79
DKC = RPW // DCH

_mesh = plsc.VectorSubcoreMesh(core_axis_name="c", subcore_axis_name="s")
_sc_params = pltpu.CompilerParams(use_tc_tiling_on_sc=False)


def _fill_rows(ref, nrows, ncols, value):
    vec = jnp.full((16,), value, jnp.float32)

    def body(i, carry):
        for k in range(ncols // 16):
            ref[i, pl.ds(16 * k, 16)] = vec
        return carry

    lax.fori_loop(0, nrows, body, 0)


@functools.partial(
    pl.kernel,
    out_type=(
        jax.ShapeDtypeStruct((NC, R, 16), jnp.float32),
        jax.ShapeDtypeStruct((NC, R, 16), jnp.float32),
    ),
    mesh=_mesh,
    compiler_params=_sc_params,
    scratch_types=[
        pltpu.VMEM((KC, CH), jnp.int32),
        pltpu.VMEM((KC, CH), jnp.int32),
        pltpu.VMEM((CH, 16), jnp.float32),
        pltpu.VMEM_SHARED((R, 16), jnp.float32),
        pltpu.VMEM_SHARED((R, 16), jnp.float32),
        pltpu.SemaphoreType.DMA,
        pltpu.SemaphoreType.DMA,
    ],
)
def _sc_degrees(src_hbm, dst_hbm, out_s_hbm, out_d_hbm,
                src_v, dst_v, stage_v, acc_s, acc_d, sem_s, sem_d):
    cid = lax.axis_index("c")
    sid = lax.axis_index("s")
    wid = cid * NS + sid

    _fill_rows(stage_v, CH, 16, 0.0)
    for k in range(DKC):
        r0 = sid * RPW + k * DCH
        pltpu.sync_copy(stage_v.at[pl.ds(0, DCH)], acc_s.at[pl.ds(r0, DCH)])
        pltpu.sync_copy(stage_v.at[pl.ds(0, DCH)], acc_d.at[pl.ds(r0, DCH)])
    _fill_rows(stage_v, CH, 16, 1.0)
    pltpu.sync_copy(src_hbm.at[wid], src_v)
    pltpu.sync_copy(dst_hbm.at[wid], dst_v)
    plsc.subcore_barrier()

    descs = [None] * KC
    for j in range(KC):
        descs[j] = (
            pltpu.async_copy(stage_v, acc_s.at[src_v.at[j]], sem_s, add=True),
            pltpu.async_copy(stage_v, acc_d.at[dst_v.at[j]], sem_d, add=True),
        )
        if j >= 2:
            descs[j - 2][0].wait()
            descs[j - 2][1].wait()
    for j in range(max(KC - 2, 0), KC):
        descs[j][0].wait()
        descs[j][1].wait()
    plsc.subcore_barrier()

    for k in range(DKC):
        r0 = sid * RPW + k * DCH
        pltpu.sync_copy(acc_s.at[pl.ds(r0, DCH)], stage_v.at[pl.ds(0, DCH)])
        pltpu.sync_copy(stage_v.at[pl.ds(0, DCH)], out_s_hbm.at[cid, pl.ds(r0, DCH)])
        pltpu.sync_copy(acc_d.at[pl.ds(r0, DCH)], stage_v.at[pl.ds(0, DCH)])
        pltpu.sync_copy(stage_v.at[pl.ds(0, DCH)], out_d_hbm.at[cid, pl.ds(r0, DCH)])


def _make_sc_agg(D):
    @functools.partial(
        pl.kernel,
        out_type=jax.ShapeDtypeStruct((NC, R, D), jnp.float32),
        mesh=_mesh,
        compiler_params=_sc_params,
        scratch_types=[
            pltpu.VMEM((KC, CH), jnp.int32),
            pltpu.VMEM((KC, CH), jnp.int32),
            pltpu.VMEM((CH, D), jnp.float32),
            pltpu.VMEM((CH, D), jnp.float32),
            pltpu.VMEM_SHARED((R, D), jnp.float32),
            pltpu.SemaphoreType.DMA,
            pltpu.SemaphoreType.DMA,
        ],
    )
    def agg(table_hbm, src_hbm, dst_hbm, out_hbm,
            src_v, dst_v, buf0, buf1, acc, sem0, sem1):
        cid = lax.axis_index("c")
        sid = lax.axis_index("s")
        wid = cid * NS + sid
        bufs = (buf0, buf1)
        sems = (sem0, sem1)

        _fill_rows(buf0, CH, D, 0.0)
        for k in range(DKC):
            r0 = sid * RPW + k * DCH
            pltpu.sync_copy(buf0.at[pl.ds(0, DCH)], acc.at[pl.ds(r0, DCH)])
        pltpu.sync_copy(src_hbm.at[wid], src_v)
        pltpu.sync_copy(dst_hbm.at[wid], dst_v)
        plsc.subcore_barrier()

        desc = [None] * KC
        desc[0] = pltpu.async_copy(table_hbm.at[src_v.at[0]], buf0, sem0)
        for j in range(KC):
            if j + 1 < KC:
                desc[j + 1] = pltpu.async_copy(
                    table_hbm.at[src_v.at[j + 1]], bufs[(j + 1) % 2],
                    sems[(j + 1) % 2])
            desc[j].wait()
            pltpu.sync_copy(bufs[j % 2], acc.at[dst_v.at[j]], add=True)
        plsc.subcore_barrier()

        for k in range(DKC):
            r0 = sid * RPW + k * DCH
            pltpu.sync_copy(acc.at[pl.ds(r0, DCH)], buf0.at[pl.ds(0, DCH)])
            pltpu.sync_copy(buf0.at[pl.ds(0, DCH)], out_hbm.at[cid, pl.ds(r0, DCH)])

    return agg


_sc_agg128 = _make_sc_agg(128)
_sc_agg16 = _make_sc_agg(16)


BLK = 632


def _norm_from(deg_ref):
    deg = deg_ref[0, :, 0] + deg_ref[1, :, 0]
    return jnp.where(deg > 0, lax.rsqrt(jnp.maximum(deg, 1.0)), 0.0)


def _tc_h1p_body(x_ref, w_ref, degs_ref, o_ref):
    h = jnp.dot(x_ref[...], w_ref[...], preferred_element_type=jnp.float32)
    o_ref[...] = h * _norm_from(degs_ref)[:, None]


def _tc_h1p(x_pad, W1, deg_s):
    return pl.pallas_call(
        _tc_h1p_body,
        grid=(R // BLK,),
        in_specs=[
            pl.BlockSpec((BLK, 128), lambda i: (i, 0)),
            pl.BlockSpec((128, 128), lambda i: (0, 0)),
            pl.BlockSpec((NC, BLK, 16), lambda i: (0, i, 0)),
        ],
        out_specs=pl.BlockSpec((BLK, 128), lambda i: (i, 0)),
        out_shape=jax.ShapeDtypeStruct((R, 128), jnp.float32),
    )(x_pad, W1, deg_s)


def _tc_mid_body(p_ref, degs_ref, degd_ref, b1_ref, w2_ref, o_ref):
    agg = p_ref[0] + p_ref[1]
    h = jax.nn.relu(agg * _norm_from(degd_ref)[:, None] + b1_ref[...])
    h2 = jnp.dot(h, w2_ref[...], preferred_element_type=jnp.float32)
    o_ref[...] = h2 * _norm_from(degs_ref)[:, None]


def _tc_mid(parts1, deg_s, deg_d, b1, W2):
    return pl.pallas_call(
        _tc_mid_body,
        grid=(R // BLK,),
        in_specs=[
            pl.BlockSpec((NC, BLK, 128), lambda i: (0, i, 0)),
            pl.BlockSpec((NC, BLK, 16), lambda i: (0, i, 0)),
            pl.BlockSpec((NC, BLK, 16), lambda i: (0, i, 0)),
            pl.BlockSpec((1, 128), lambda i: (0, 0)),
            pl.BlockSpec((128, 16), lambda i: (0, 0)),
        ],
        out_specs=pl.BlockSpec((BLK, 16), lambda i: (i, 0)),
        out_shape=jax.ShapeDtypeStruct((R, 16), jnp.float32),
    )(parts1, deg_s, deg_d, b1, W2)


def _tc_softmax_body(p_ref, degd_ref, b2_ref, o_ref):
    agg = p_ref[0] + p_ref[1]
    z = agg * _norm_from(degd_ref)[:, None] + b2_ref[...]
    z = z - jnp.max(z, axis=1, keepdims=True)
    ez = jnp.exp(z)
    o_ref[...] = ez / jnp.sum(ez, axis=1, keepdims=True)


def _tc_softmax(parts2, deg_d, b2):
    return pl.pallas_call(
        _tc_softmax_body,
        grid=(R // BLK,),
        in_specs=[
            pl.BlockSpec((NC, BLK, 16), lambda i: (0, i, 0)),
            pl.BlockSpec((NC, BLK, 16), lambda i: (0, i, 0)),
            pl.BlockSpec((1, 16), lambda i: (0, 0)),
        ],
        out_specs=pl.BlockSpec((BLK, 16), lambda i: (i, 0)),
        out_shape=jax.ShapeDtypeStruct((R, 16), jnp.float32),
    )(parts2, deg_d, b2)


def kernel(edge_index, inputs, W1, b1, W2, b2):
    src = edge_index[0].astype(jnp.int32)
    dst = edge_index[1].astype(jnp.int32)
    ppw = EPW - E // NW
    pad = PAD_ROW + (jnp.arange(NW * ppw, dtype=jnp.int32) % (R - N))
    pad = pad.reshape(NW, ppw)
    src3 = jnp.concatenate([src.reshape(NW, E // NW), pad], axis=1)
    src3 = src3.reshape(NW, KC, CH)
    dst3 = jnp.concatenate([dst.reshape(NW, E // NW), pad], axis=1)
    dst3 = dst3.reshape(NW, KC, CH)

    deg_s, deg_d = _sc_degrees(src3, dst3)

    x_pad = jnp.pad(inputs, ((0, R - N), (0, 0)))
    h1p = _tc_h1p(x_pad, W1, deg_s)
    parts1 = _sc_agg128(h1p, src3, dst3)
    h2p = _tc_mid(parts1, deg_s, deg_d, b1.reshape(1, 128), W2)
    parts2 = _sc_agg16(h2p, src3, dst3)
    out = _tc_softmax(parts2, deg_d, b2.reshape(1, 16))
    return out[:N]


# speedup vs baseline: 22.5541x; 1.0156x over previous
import functools

import jax
import jax.numpy as jnp
from jax import lax
from jax.experimental import pallas as pl
from jax.experimental.pallas import tpu as pltpu
from jax.experimental.pallas import tpu_sc as plsc

N = 10000
R = 10240
E = 320000
NC = 2
NS = 16
NW = NC * NS
CH = 100
KC = 100
RPW = R // NS
DCH = 80
DKC = RPW // DCH
RP8 = R // 8

_mesh = plsc.VectorSubcoreMesh(core_axis_name="c", subcore_axis_name="s")
_sc_params = pltpu.CompilerParams(use_tc_tiling_on_sc=False,
                                  needs_layout_passes=False)


def _fill_rows(ref, nrows, ncols, value):
    vec = jnp.full((16,), value, jnp.float32)

    def body(i, carry):
        for k in range(ncols // 16):
            ref[i, pl.ds(16 * k, 16)] = vec
        return carry

    lax.fori_loop(0, nrows, body, 0)


@functools.partial(
    pl.kernel,
    out_type=(
        jax.ShapeDtypeStruct((NC, R), jnp.float32),
        jax.ShapeDtypeStruct((NC, R), jnp.float32),
        jax.ShapeDtypeStruct((NC, RP8, 128), jnp.float32),
    ),
    mesh=_mesh,
    compiler_params=_sc_params,
    scratch_types=[
        pltpu.VMEM((KC, CH), jnp.int32),
        pltpu.VMEM((KC, CH), jnp.int32),
        pltpu.VMEM((CH, 16), jnp.float32),
        pltpu.VMEM((DCH, 16), jnp.float32),
        pltpu.VMEM((RPW,), jnp.float32),
        pltpu.VMEM((DCH * 16 // 128, 128), jnp.float32),
        pltpu.VMEM_SHARED((R, 16), jnp.float32),
        pltpu.VMEM_SHARED((R, 16), jnp.float32),
        pltpu.SemaphoreType.DMA,
        pltpu.SemaphoreType.DMA,
    ],
)
def _sc_degrees(src_hbm, dst_hbm, out_s_hbm, out_d_hbm, out_dp_hbm,
                src_v, dst_v, stage_v, ex_v, cvec, pview,
                acc_s, acc_d, sem_s, sem_d):
    cid = lax.axis_index("c")
    sid = lax.axis_index("s")
    wid = cid * NS + sid

    _fill_rows(stage_v, DCH, 16, 0.0)
    for k in range(DKC):
        r0 = sid * RPW + k * DCH
        pltpu.sync_copy(stage_v.at[pl.ds(0, DCH)], acc_s.at[pl.ds(r0, DCH)])
        pltpu.sync_copy(stage_v.at[pl.ds(0, DCH)], acc_d.at[pl.ds(r0, DCH)])
    _fill_rows(stage_v, CH, 16, 1.0)
    pltpu.sync_copy(src_hbm.at[wid], src_v)
    pltpu.sync_copy(dst_hbm.at[wid], dst_v)
    plsc.subcore_barrier()

    descs = [None] * KC
    for j in range(KC):
        descs[j] = (
            pltpu.async_copy(stage_v.at[pl.ds(0, CH)], acc_s.at[src_v.at[j]],
                             sem_s, add=True),
            pltpu.async_copy(stage_v.at[pl.ds(0, CH)], acc_d.at[dst_v.at[j]],
                             sem_d, add=True),
        )
        if j >= 2:
            descs[j - 2][0].wait()
            descs[j - 2][1].wait()
    for j in range(max(KC - 2, 0), KC):
        descs[j][0].wait()
        descs[j][1].wait()
    plsc.subcore_barrier()

    iota = lax.iota(jnp.int32, 16)
    zcol = jnp.zeros((16,), jnp.int32)
    for acc, out_hbm, dp in ((acc_s, out_s_hbm, None), (acc_d, out_d_hbm, out_dp_hbm)):
        for k in range(DKC):
            r0 = sid * RPW + k * DCH
            pltpu.sync_copy(acc.at[pl.ds(r0, DCH)], ex_v)
            for m in range(DCH // 16):
                vals = plsc.load_gather(ex_v, [iota + 16 * m, zcol])
                cvec[pl.ds(k * DCH + 16 * m, 16)] = vals
            if dp is not None:
                for r in range(DCH):
                    pview[r // 8, pl.ds((r % 8) * 16, 16)] = ex_v[r, :]
                p0 = r0 * 16 // 128
                pltpu.sync_copy(pview, dp.at[cid, pl.ds(p0, DCH * 16 // 128)])
        pltpu.sync_copy(cvec, out_hbm.at[cid, pl.ds(sid * RPW, RPW)])


def _make_sc_agg(D, packed):
    out_shape = (NC, RP8, 128) if packed else (NC, R, D)
    pview_types = [pltpu.VMEM((DCH * D // 128, 128), jnp.float32)] if packed else []

    @functools.partial(
        pl.kernel,
        out_type=jax.ShapeDtypeStruct(out_shape, jnp.float32),
        mesh=_mesh,
        compiler_params=_sc_params,
        scratch_types=[
            pltpu.VMEM((KC, CH), jnp.int32),
            pltpu.VMEM((KC, CH), jnp.int32),
            pltpu.VMEM((CH, D), jnp.float32),
            pltpu.VMEM((CH, D), jnp.float32),
            pltpu.VMEM_SHARED((R, D), jnp.float32),
            pltpu.SemaphoreType.DMA,
            pltpu.SemaphoreType.DMA,
        ] + pview_types,
    )
    def agg(table_hbm, src_hbm, dst_hbm, out_hbm,
            src_v, dst_v, buf0, buf1, acc, sem0, sem1, *maybe_pview):
        cid = lax.axis_index("c")
        sid = lax.axis_index("s")
        wid = cid * NS + sid
        bufs = (buf0, buf1)
        sems = (sem0, sem1)

        _fill_rows(buf0, DCH, D, 0.0)
        for k in range(DKC):
            r0 = sid * RPW + k * DCH
            pltpu.sync_copy(buf0.at[pl.ds(0, DCH)], acc.at[pl.ds(r0, DCH)])
        pltpu.sync_copy(src_hbm.at[wid], src_v)
        pltpu.sync_copy(dst_hbm.at[wid], dst_v)
        plsc.subcore_barrier()

        desc = [None] * KC
        desc[0] = pltpu.async_copy(table_hbm.at[src_v.at[0]], buf0, sem0)
        for j in range(KC):
            if j + 1 < KC:
                desc[j + 1] = pltpu.async_copy(
                    table_hbm.at[src_v.at[j + 1]], bufs[(j + 1) % 2],
                    sems[(j + 1) % 2])
            desc[j].wait()
            pltpu.sync_copy(bufs[j % 2], acc.at[dst_v.at[j]], add=True)
        plsc.subcore_barrier()

        for k in range(DKC):
            r0 = sid * RPW + k * DCH
            pltpu.sync_copy(acc.at[pl.ds(r0, DCH)], buf0.at[pl.ds(0, DCH)])
            if packed:
                pview = maybe_pview[0]
                for r in range(DCH):
                    pview[r // 8, pl.ds((r % 8) * 16, 16)] = buf0[r, :]
                p0 = (sid * RPW + k * DCH) * D // 128
                pltpu.sync_copy(pview, out_hbm.at[cid, pl.ds(p0, DCH * D // 128)])
            else:
                pltpu.sync_copy(buf0.at[pl.ds(0, DCH)],
                                out_hbm.at[cid, pl.ds(r0, DCH)])

    return agg


_sc_agg128 = _make_sc_agg(128, packed=False)
_sc_agg16 = _make_sc_agg(16, packed=True)


BLK = 512
BLK2 = 512


def _norm_from(deg_ref, blk):
    deg = deg_ref[0, :] + deg_ref[1, :]
    norm = jnp.where(deg > 0, lax.rsqrt(jnp.maximum(deg, 1.0)), 0.0)
    return norm.reshape(blk, 1)


def _tc_h1p_body(x_ref, w_ref, degs_ref, o_ref):
    h = jnp.dot(x_ref[...], w_ref[...], preferred_element_type=jnp.float32)
    o_ref[...] = h * _norm_from(degs_ref, BLK)


def _tc_h1p(x_pad, W1, deg_s):
    return pl.pallas_call(
        _tc_h1p_body,
        grid=(R // BLK,),
        in_specs=[
            pl.BlockSpec((BLK, 128), lambda i: (i, 0)),
            pl.BlockSpec((128, 128), lambda i: (0, 0)),
            pl.BlockSpec((NC, BLK), lambda i: (0, i)),
        ],
        out_specs=pl.BlockSpec((BLK, 128), lambda i: (i, 0)),
        out_shape=jax.ShapeDtypeStruct((R, 128), jnp.float32),
    )(x_pad, W1, deg_s)


def _tc_mid_body(p_ref, degs_ref, degd_ref, b1_ref, w2_ref, o_ref):
    agg = p_ref[0] + p_ref[1]
    h = jax.nn.relu(agg * _norm_from(degd_ref, BLK) + b1_ref[...])
    h2 = jnp.dot(h, w2_ref[...], preferred_element_type=jnp.float32)
    o_ref[...] = h2 * _norm_from(degs_ref, BLK)


def _tc_mid(parts1, deg_s, deg_d, b1, W2):
    return pl.pallas_call(
        _tc_mid_body,
        grid=(R // BLK,),
        in_specs=[
            pl.BlockSpec((NC, BLK, 128), lambda i: (0, i, 0)),
            pl.BlockSpec((NC, BLK), lambda i: (0, i)),
            pl.BlockSpec((NC, BLK), lambda i: (0, i)),
            pl.BlockSpec((1, 128), lambda i: (0, 0)),
            pl.BlockSpec((128, 16), lambda i: (0, 0)),
        ],
        out_specs=pl.BlockSpec((BLK, 16), lambda i: (i, 0)),
        out_shape=jax.ShapeDtypeStruct((R, 16), jnp.float32),
    )(parts1, deg_s, deg_d, b1, W2)


PBLK = BLK2 * 16 // 128


def _tc_softmax_body(p_ref, degdp_ref, b2p_ref, o_ref):
    agg = p_ref[0] + p_ref[1]
    deg = degdp_ref[0] + degdp_ref[1]
    norm = jnp.where(deg > 0, lax.rsqrt(jnp.maximum(deg, 1.0)), 0.0)
    z = agg * norm + b2p_ref[...]
    outs = []
    for g in range(8):
        zg = z[:, 16 * g:16 * (g + 1)]
        eg = jnp.exp(zg - jnp.max(zg, axis=1, keepdims=True))
        outs.append(eg / jnp.sum(eg, axis=1, keepdims=True))
    o_ref[...] = jnp.concatenate(outs, axis=1)


def _tc_softmax(parts2, deg_dp, b2p):
    return pl.pallas_call(
        _tc_softmax_body,
        grid=(R // BLK2,),
        in_specs=[
            pl.BlockSpec((NC, PBLK, 128), lambda i: (0, i, 0)),
            pl.BlockSpec((NC, PBLK, 128), lambda i: (0, i, 0)),
            pl.BlockSpec((1, 128), lambda i: (0, 0)),
        ],
        out_specs=pl.BlockSpec((PBLK, 128), lambda i: (i, 0)),
        out_shape=jax.ShapeDtypeStruct((RP8, 128), jnp.float32),
    )(parts2, deg_dp, b2p)


def kernel(edge_index, inputs, W1, b1, W2, b2):
    src3 = edge_index[0].astype(jnp.int32).reshape(NW, KC, CH)
    dst3 = edge_index[1].astype(jnp.int32).reshape(NW, KC, CH)

    deg_s, deg_d, deg_dp = _sc_degrees(src3, dst3)

    x_pad = jnp.pad(inputs, ((0, R - N), (0, 0)))
    h1p = _tc_h1p(x_pad, W1, deg_s)
    parts1 = _sc_agg128(h1p, src3, dst3)
    h2p = _tc_mid(parts1, deg_s, deg_d, b1.reshape(1, 128), W2)
    parts2 = _sc_agg16(h2p, src3, dst3)
    b2p = jnp.tile(b2.reshape(1, 16), (1, 8))
    out = _tc_softmax(parts2, deg_dp, b2p)
    return out.reshape(R, 16)[:N]


# speedup vs baseline: 23.4155x; 1.0382x over previous
import functools

import jax
import jax.numpy as jnp
from jax import lax
from jax.experimental import pallas as pl
from jax.experimental.pallas import tpu as pltpu
from jax.experimental.pallas import tpu_sc as plsc

N = 10000
R = 10240
E = 320000
NC = 2
NS = 16
NW = NC * NS
CH = 100
KC = 100
RPW = R // NS
DCH = 80
DKC = RPW // DCH
RP8 = R // 8

_mesh = plsc.VectorSubcoreMesh(core_axis_name="c", subcore_axis_name="s")
_sc_params = pltpu.CompilerParams(use_tc_tiling_on_sc=False,
                                  needs_layout_passes=False)


def _fill_rows(ref, nrows, ncols, value):
    vec = jnp.full((16,), value, jnp.float32)

    def body(i, carry):
        for k in range(ncols // 16):
            ref[i, pl.ds(16 * k, 16)] = vec
        return carry

    lax.fori_loop(0, nrows, body, 0)


@functools.partial(
    pl.kernel,
    out_type=(
        jax.ShapeDtypeStruct((NC, R), jnp.float32),
        jax.ShapeDtypeStruct((NC, R), jnp.float32),
        jax.ShapeDtypeStruct((NC, RP8, 128), jnp.float32),
    ),
    mesh=_mesh,
    compiler_params=_sc_params,
    scratch_types=[
        pltpu.VMEM((KC, CH), jnp.int32),
        pltpu.VMEM((KC, CH), jnp.int32),
        pltpu.VMEM((CH, 16), jnp.float32),
        pltpu.VMEM((DCH, 16), jnp.float32),
        pltpu.VMEM((RPW,), jnp.float32),
        pltpu.VMEM((DCH * 16 // 128, 128), jnp.float32),
        pltpu.VMEM_SHARED((R, 16), jnp.float32),
        pltpu.VMEM_SHARED((R, 16), jnp.float32),
        pltpu.SemaphoreType.DMA,
        pltpu.SemaphoreType.DMA,
    ],
)
def _sc_degrees(edges_hbm, out_s_hbm, out_d_hbm, out_dp_hbm,
                src_v, dst_v, stage_v, ex_v, cvec, pview,
                acc_s, acc_d, sem_s, sem_d):
    cid = lax.axis_index("c")
    sid = lax.axis_index("s")
    wid = cid * NS + sid

    _fill_rows(stage_v, DCH, 16, 0.0)
    for k in range(DKC):
        r0 = sid * RPW + k * DCH
        pltpu.sync_copy(stage_v.at[pl.ds(0, DCH)], acc_s.at[pl.ds(r0, DCH)])
        pltpu.sync_copy(stage_v.at[pl.ds(0, DCH)], acc_d.at[pl.ds(r0, DCH)])
    _fill_rows(stage_v, CH, 16, 1.0)
    pltpu.sync_copy(edges_hbm.at[0, wid], src_v)
    pltpu.sync_copy(edges_hbm.at[1, wid], dst_v)
    plsc.subcore_barrier()

    descs = [None] * KC
    for j in range(KC):
        descs[j] = (
            pltpu.async_copy(stage_v.at[pl.ds(0, CH)], acc_s.at[src_v.at[j]],
                             sem_s, add=True),
            pltpu.async_copy(stage_v.at[pl.ds(0, CH)], acc_d.at[dst_v.at[j]],
                             sem_d, add=True),
        )
        if j >= 2:
            descs[j - 2][0].wait()
            descs[j - 2][1].wait()
    for j in range(max(KC - 2, 0), KC):
        descs[j][0].wait()
        descs[j][1].wait()
    plsc.subcore_barrier()

    iota = lax.iota(jnp.int32, 16)
    zcol = jnp.zeros((16,), jnp.int32)
    for acc, out_hbm, dp in ((acc_s, out_s_hbm, None), (acc_d, out_d_hbm, out_dp_hbm)):
        for k in range(DKC):
            r0 = sid * RPW + k * DCH
            pltpu.sync_copy(acc.at[pl.ds(r0, DCH)], ex_v)
            for m in range(DCH // 16):
                vals = plsc.load_gather(ex_v, [iota + 16 * m, zcol])
                cvec[pl.ds(k * DCH + 16 * m, 16)] = vals
            if dp is not None:
                for r in range(DCH):
                    pview[r // 8, pl.ds((r % 8) * 16, 16)] = ex_v[r, :]
                p0 = r0 * 16 // 128
                pltpu.sync_copy(pview, dp.at[cid, pl.ds(p0, DCH * 16 // 128)])
        pltpu.sync_copy(cvec, out_hbm.at[cid, pl.ds(sid * RPW, RPW)])


def _make_sc_agg(D, packed):
    out_shape = (NC, RP8, 128) if packed else (NC, R, D)
    pview_types = [pltpu.VMEM((DCH * D // 128, 128), jnp.float32)] if packed else []

    @functools.partial(
        pl.kernel,
        out_type=jax.ShapeDtypeStruct(out_shape, jnp.float32),
        mesh=_mesh,
        compiler_params=_sc_params,
        scratch_types=[
            pltpu.VMEM((KC, CH), jnp.int32),
            pltpu.VMEM((KC, CH), jnp.int32),
            pltpu.VMEM((CH, D), jnp.float32),
            pltpu.VMEM((CH, D), jnp.float32),
            pltpu.VMEM_SHARED((R, D), jnp.float32),
            pltpu.SemaphoreType.DMA,
            pltpu.SemaphoreType.DMA,
        ] + pview_types,
    )
    def agg(table_hbm, edges_hbm, out_hbm,
            src_v, dst_v, buf0, buf1, acc, sem0, sem1, *maybe_pview):
        cid = lax.axis_index("c")
        sid = lax.axis_index("s")
        wid = cid * NS + sid
        bufs = (buf0, buf1)
        sems = (sem0, sem1)

        _fill_rows(buf0, DCH, D, 0.0)
        for k in range(DKC):
            r0 = sid * RPW + k * DCH
            pltpu.sync_copy(buf0.at[pl.ds(0, DCH)], acc.at[pl.ds(r0, DCH)])
        pltpu.sync_copy(edges_hbm.at[0, wid], src_v)
        pltpu.sync_copy(edges_hbm.at[1, wid], dst_v)
        plsc.subcore_barrier()

        desc = [None] * KC
        desc[0] = pltpu.async_copy(table_hbm.at[src_v.at[0]], buf0, sem0)
        for j in range(KC):
            if j + 1 < KC:
                desc[j + 1] = pltpu.async_copy(
                    table_hbm.at[src_v.at[j + 1]], bufs[(j + 1) % 2],
                    sems[(j + 1) % 2])
            desc[j].wait()
            pltpu.sync_copy(bufs[j % 2], acc.at[dst_v.at[j]], add=True)
        plsc.subcore_barrier()

        for k in range(DKC):
            r0 = sid * RPW + k * DCH
            pltpu.sync_copy(acc.at[pl.ds(r0, DCH)], buf0.at[pl.ds(0, DCH)])
            if packed:
                pview = maybe_pview[0]
                for r in range(DCH):
                    pview[r // 8, pl.ds((r % 8) * 16, 16)] = buf0[r, :]
                p0 = (sid * RPW + k * DCH) * D // 128
                pltpu.sync_copy(pview, out_hbm.at[cid, pl.ds(p0, DCH * D // 128)])
            else:
                pltpu.sync_copy(buf0.at[pl.ds(0, DCH)],
                                out_hbm.at[cid, pl.ds(r0, DCH)])

    return agg


_sc_agg128 = _make_sc_agg(128, packed=False)
_sc_agg16 = _make_sc_agg(16, packed=True)


BLK = 512
BLK2 = 512


def _norm_from(deg_ref, blk):
    deg = deg_ref[0, :] + deg_ref[1, :]
    norm = jnp.where(deg > 0, lax.rsqrt(jnp.maximum(deg, 1.0)), 0.0)
    return norm.reshape(blk, 1)


def _tc_h1p_body(x_ref, w_ref, degs_ref, o_ref):
    h = jnp.dot(x_ref[...], w_ref[...], preferred_element_type=jnp.float32)
    o_ref[...] = h * _norm_from(degs_ref, BLK)


def _tc_h1p(x_pad, W1, deg_s):
    return pl.pallas_call(
        _tc_h1p_body,
        grid=(R // BLK,),
        in_specs=[
            pl.BlockSpec((BLK, 128), lambda i: (i, 0)),
            pl.BlockSpec((128, 128), lambda i: (0, 0)),
            pl.BlockSpec((NC, BLK), lambda i: (0, i)),
        ],
        out_specs=pl.BlockSpec((BLK, 128), lambda i: (i, 0)),
        out_shape=jax.ShapeDtypeStruct((R, 128), jnp.float32),
    )(x_pad, W1, deg_s)


def _tc_mid_body(p_ref, degs_ref, degd_ref, b1_ref, w2_ref, o_ref):
    agg = p_ref[0] + p_ref[1]
    h = jax.nn.relu(agg * _norm_from(degd_ref, BLK) + b1_ref[...])
    h2 = jnp.dot(h, w2_ref[...], preferred_element_type=jnp.float32)
    o_ref[...] = h2 * _norm_from(degs_ref, BLK)


def _tc_mid(parts1, deg_s, deg_d, b1, W2):
    return pl.pallas_call(
        _tc_mid_body,
        grid=(R // BLK,),
        in_specs=[
            pl.BlockSpec((NC, BLK, 128), lambda i: (0, i, 0)),
            pl.BlockSpec((NC, BLK), lambda i: (0, i)),
            pl.BlockSpec((NC, BLK), lambda i: (0, i)),
            pl.BlockSpec((1, 128), lambda i: (0, 0)),
            pl.BlockSpec((128, 16), lambda i: (0, 0)),
        ],
        out_specs=pl.BlockSpec((BLK, 16), lambda i: (i, 0)),
        out_shape=jax.ShapeDtypeStruct((R, 16), jnp.float32),
    )(parts1, deg_s, deg_d, b1, W2)


PBLK = BLK2 * 16 // 128


def _tc_softmax_body(p_ref, degdp_ref, b2p_ref, gmask_ref, o_ref):
    agg = p_ref[0] + p_ref[1]
    deg = degdp_ref[0] + degdp_ref[1]
    norm = jnp.where(deg > 0, lax.rsqrt(jnp.maximum(deg, 1.0)), 0.0)
    z = agg * norm + b2p_ref[...]
    ez = jnp.exp(z - jnp.max(z, axis=1, keepdims=True))
    s = jnp.dot(ez, gmask_ref[...], preferred_element_type=jnp.float32)
    o_ref[...] = ez / s


def _tc_softmax(parts2, deg_dp, b2p, gmask):
    return pl.pallas_call(
        _tc_softmax_body,
        grid=(R // BLK2,),
        in_specs=[
            pl.BlockSpec((NC, PBLK, 128), lambda i: (0, i, 0)),
            pl.BlockSpec((NC, PBLK, 128), lambda i: (0, i, 0)),
            pl.BlockSpec((1, 128), lambda i: (0, 0)),
            pl.BlockSpec((128, 128), lambda i: (0, 0)),
        ],
        out_specs=pl.BlockSpec((PBLK, 128), lambda i: (i, 0)),
        out_shape=jax.ShapeDtypeStruct((RP8, 128), jnp.float32),
    )(parts2, deg_dp, b2p, gmask)


def kernel(edge_index, inputs, W1, b1, W2, b2):
    edges4 = edge_index.astype(jnp.int32).reshape(2, NW, KC, CH)

    deg_s, deg_d, deg_dp = _sc_degrees(edges4)

    x_pad = jnp.pad(inputs, ((0, R - N), (0, 0)))
    h1p = _tc_h1p(x_pad, W1, deg_s)
    parts1 = _sc_agg128(h1p, edges4)
    h2p = _tc_mid(parts1, deg_s, deg_d, b1.reshape(1, 128), W2)
    parts2 = _sc_agg16(h2p, edges4)
    b2p = jnp.tile(b2.reshape(1, 16), (1, 8))
    gmask = jnp.kron(jnp.eye(8, dtype=jnp.float32),
                     jnp.ones((16, 16), jnp.float32))
    out = _tc_softmax(parts2, deg_dp, b2p, gmask)
    return out.reshape(R, 16)[:N]


# speedup vs baseline: 24.2186x; 1.0343x over previous
import functools

import jax
import jax.numpy as jnp
from jax import lax
from jax.experimental import pallas as pl
from jax.experimental.pallas import tpu as pltpu
from jax.experimental.pallas import tpu_sc as plsc

N = 10000
R = 10240
E = 320000
NC = 2
NS = 16
NW = NC * NS
CH = 100
KC = 100
RPW = R // NS
DCH = 80
DKC = RPW // DCH
RP8 = R // 8

_mesh = plsc.VectorSubcoreMesh(core_axis_name="c", subcore_axis_name="s")
_sc_params = pltpu.CompilerParams(use_tc_tiling_on_sc=False,
                                  needs_layout_passes=False)


def _fill_rows(ref, nrows, ncols, value):
    vec = jnp.full((16,), value, jnp.float32)

    def body(i, carry):
        for k in range(ncols // 16):
            ref[i, pl.ds(16 * k, 16)] = vec
        return carry

    lax.fori_loop(0, nrows, body, 0)


@functools.partial(
    pl.kernel,
    out_type=(
        jax.ShapeDtypeStruct((NC, R), jnp.float32),
        jax.ShapeDtypeStruct((NC, R), jnp.float32),
        jax.ShapeDtypeStruct((NC, RP8, 128), jnp.float32),
    ),
    mesh=_mesh,
    compiler_params=_sc_params,
    scratch_types=[
        pltpu.VMEM((KC, CH), jnp.int32),
        pltpu.VMEM((KC, CH), jnp.int32),
        pltpu.VMEM((CH, 16), jnp.float32),
        pltpu.VMEM((DCH, 16), jnp.float32),
        pltpu.VMEM((RPW,), jnp.float32),
        pltpu.VMEM((DCH * 16 // 128, 128), jnp.float32),
        pltpu.VMEM_SHARED((R, 16), jnp.float32),
        pltpu.VMEM_SHARED((R, 16), jnp.float32),
        pltpu.SemaphoreType.DMA,
        pltpu.SemaphoreType.DMA,
    ],
)
def _sc_degrees(edges_hbm, out_s_hbm, out_d_hbm, out_dp_hbm,
                src_v, dst_v, stage_v, ex_v, cvec, pview,
                acc_s, acc_d, sem_s, sem_d):
    cid = lax.axis_index("c")
    sid = lax.axis_index("s")
    wid = cid * NS + sid

    _fill_rows(stage_v, DCH, 16, 0.0)
    for k in range(DKC):
        r0 = sid * RPW + k * DCH
        pltpu.sync_copy(stage_v.at[pl.ds(0, DCH)], acc_s.at[pl.ds(r0, DCH)])
        pltpu.sync_copy(stage_v.at[pl.ds(0, DCH)], acc_d.at[pl.ds(r0, DCH)])
    _fill_rows(stage_v, CH, 16, 1.0)
    pltpu.sync_copy(edges_hbm.at[0, wid], src_v)
    pltpu.sync_copy(edges_hbm.at[1, wid], dst_v)
    plsc.subcore_barrier()

    descs = [None] * KC
    for j in range(KC):
        descs[j] = (
            pltpu.async_copy(stage_v.at[pl.ds(0, CH)], acc_s.at[src_v.at[j]],
                             sem_s, add=True),
            pltpu.async_copy(stage_v.at[pl.ds(0, CH)], acc_d.at[dst_v.at[j]],
                             sem_d, add=True),
        )
        if j >= 2:
            descs[j - 2][0].wait()
            descs[j - 2][1].wait()
    for j in range(max(KC - 2, 0), KC):
        descs[j][0].wait()
        descs[j][1].wait()
    plsc.subcore_barrier()

    iota = lax.iota(jnp.int32, 16)
    zcol = jnp.zeros((16,), jnp.int32)
    for acc, out_hbm, dp in ((acc_s, out_s_hbm, None), (acc_d, out_d_hbm, out_dp_hbm)):
        for k in range(DKC):
            r0 = sid * RPW + k * DCH
            pltpu.sync_copy(acc.at[pl.ds(r0, DCH)], ex_v)
            for m in range(DCH // 16):
                vals = plsc.load_gather(ex_v, [iota + 16 * m, zcol])
                cvec[pl.ds(k * DCH + 16 * m, 16)] = vals
            if dp is not None:
                for r in range(DCH):
                    pview[r // 8, pl.ds((r % 8) * 16, 16)] = ex_v[r, :]
                p0 = r0 * 16 // 128
                pltpu.sync_copy(pview, dp.at[cid, pl.ds(p0, DCH * 16 // 128)])
        pltpu.sync_copy(cvec, out_hbm.at[cid, pl.ds(sid * RPW, RPW)])


def _make_sc_agg(D, packed):
    out_shape = (NC, RP8, 128) if packed else (NC, R, D)
    pview_types = [pltpu.VMEM((DCH * D // 128, 128), jnp.float32)] if packed else []

    @functools.partial(
        pl.kernel,
        out_type=jax.ShapeDtypeStruct(out_shape, jnp.float32),
        mesh=_mesh,
        compiler_params=_sc_params,
        scratch_types=[
            pltpu.VMEM((KC, CH), jnp.int32),
            pltpu.VMEM((KC, CH), jnp.int32),
            pltpu.VMEM((CH, D), jnp.float32),
            pltpu.VMEM((CH, D), jnp.float32),
            pltpu.VMEM_SHARED((R, D), jnp.float32),
            pltpu.SemaphoreType.DMA,
            pltpu.SemaphoreType.DMA,
        ] + pview_types,
    )
    def agg(table_hbm, edges_hbm, out_hbm,
            src_v, dst_v, buf0, buf1, acc, sem0, sem1, *maybe_pview):
        cid = lax.axis_index("c")
        sid = lax.axis_index("s")
        wid = cid * NS + sid
        bufs = (buf0, buf1)
        sems = (sem0, sem1)

        _fill_rows(buf0, DCH, D, 0.0)
        for k in range(DKC):
            r0 = sid * RPW + k * DCH
            pltpu.sync_copy(buf0.at[pl.ds(0, DCH)], acc.at[pl.ds(r0, DCH)])
        pltpu.sync_copy(edges_hbm.at[0, wid], src_v)
        pltpu.sync_copy(edges_hbm.at[1, wid], dst_v)
        plsc.subcore_barrier()

        desc = [None] * KC
        desc[0] = pltpu.async_copy(table_hbm.at[src_v.at[0]], buf0, sem0)
        for j in range(KC):
            if j + 1 < KC:
                desc[j + 1] = pltpu.async_copy(
                    table_hbm.at[src_v.at[j + 1]], bufs[(j + 1) % 2],
                    sems[(j + 1) % 2])
            desc[j].wait()
            pltpu.sync_copy(bufs[j % 2], acc.at[dst_v.at[j]], add=True)
        plsc.subcore_barrier()

        for k in range(DKC):
            r0 = sid * RPW + k * DCH
            pltpu.sync_copy(acc.at[pl.ds(r0, DCH)], buf0.at[pl.ds(0, DCH)])
            if packed:
                pview = maybe_pview[0]
                for r in range(DCH):
                    pview[r // 8, pl.ds((r % 8) * 16, 16)] = buf0[r, :]
                p0 = (sid * RPW + k * DCH) * D // 128
                pltpu.sync_copy(pview, out_hbm.at[cid, pl.ds(p0, DCH * D // 128)])
            else:
                pltpu.sync_copy(buf0.at[pl.ds(0, DCH)],
                                out_hbm.at[cid, pl.ds(r0, DCH)])

    return agg


_sc_agg128 = _make_sc_agg(128, packed=False)
_sc_agg16 = _make_sc_agg(16, packed=True)


BLK = 1024
BLK2 = 512


def _norm_from(deg_ref, blk):
    deg = deg_ref[0, :] + deg_ref[1, :]
    norm = jnp.where(deg > 0, lax.rsqrt(jnp.maximum(deg, 1.0)), 0.0)
    return norm.reshape(blk, 1)


def _tc_h1p_body(x_ref, w_ref, degs_ref, o_ref):
    h = jnp.dot(x_ref[...], w_ref[...], preferred_element_type=jnp.float32)
    o_ref[...] = h * _norm_from(degs_ref, BLK)


def _tc_h1p(x_pad, W1, deg_s):
    return pl.pallas_call(
        _tc_h1p_body,
        grid=(R // BLK,),
        in_specs=[
            pl.BlockSpec((BLK, 128), lambda i: (i, 0)),
            pl.BlockSpec((128, 128), lambda i: (0, 0)),
            pl.BlockSpec((NC, BLK), lambda i: (0, i)),
        ],
        out_specs=pl.BlockSpec((BLK, 128), lambda i: (i, 0)),
        out_shape=jax.ShapeDtypeStruct((R, 128), jnp.float32),
    )(x_pad, W1, deg_s)


def _tc_mid_body(p_ref, degs_ref, degd_ref, b1_ref, w2_ref, o_ref):
    agg = p_ref[0] + p_ref[1]
    h = jax.nn.relu(agg * _norm_from(degd_ref, BLK) + b1_ref[...])
    h2 = jnp.dot(h, w2_ref[...], preferred_element_type=jnp.float32)
    o_ref[...] = h2 * _norm_from(degs_ref, BLK)


def _tc_mid(parts1, deg_s, deg_d, b1, W2):
    return pl.pallas_call(
        _tc_mid_body,
        grid=(R // BLK,),
        in_specs=[
            pl.BlockSpec((NC, BLK, 128), lambda i: (0, i, 0)),
            pl.BlockSpec((NC, BLK), lambda i: (0, i)),
            pl.BlockSpec((NC, BLK), lambda i: (0, i)),
            pl.BlockSpec((1, 128), lambda i: (0, 0)),
            pl.BlockSpec((128, 16), lambda i: (0, 0)),
        ],
        out_specs=pl.BlockSpec((BLK, 16), lambda i: (i, 0)),
        out_shape=jax.ShapeDtypeStruct((R, 16), jnp.float32),
    )(parts1, deg_s, deg_d, b1, W2)


PBLK = BLK2 * 16 // 128


def _tc_softmax_body(p_ref, degdp_ref, b2p_ref, gmask_ref, o_ref):
    agg = p_ref[0] + p_ref[1]
    deg = degdp_ref[0] + degdp_ref[1]
    norm = jnp.where(deg > 0, lax.rsqrt(jnp.maximum(deg, 1.0)), 0.0)
    z = agg * norm + b2p_ref[...]
    ez = jnp.exp(z - jnp.max(z, axis=1, keepdims=True))
    s = jnp.dot(ez, gmask_ref[...], preferred_element_type=jnp.float32,
                precision=lax.Precision.HIGHEST)
    o_ref[...] = ez / s


def _tc_softmax(parts2, deg_dp, b2p, gmask):
    return pl.pallas_call(
        _tc_softmax_body,
        grid=(R // BLK2,),
        in_specs=[
            pl.BlockSpec((NC, PBLK, 128), lambda i: (0, i, 0)),
            pl.BlockSpec((NC, PBLK, 128), lambda i: (0, i, 0)),
            pl.BlockSpec((1, 128), lambda i: (0, 0)),
            pl.BlockSpec((128, 128), lambda i: (0, 0)),
        ],
        out_specs=pl.BlockSpec((PBLK, 128), lambda i: (i, 0)),
        out_shape=jax.ShapeDtypeStruct((RP8, 128), jnp.float32),
    )(parts2, deg_dp, b2p, gmask)


def kernel(edge_index, inputs, W1, b1, W2, b2):
    edges4 = edge_index.astype(jnp.int32).reshape(2, NW, KC, CH)

    deg_s, deg_d, deg_dp = _sc_degrees(edges4)

    x_pad = jnp.pad(inputs, ((0, R - N), (0, 0)))
    h1p = _tc_h1p(x_pad, W1, deg_s)
    parts1 = _sc_agg128(h1p, edges4)
    h2p = _tc_mid(parts1, deg_s, deg_d, b1.reshape(1, 128), W2)
    parts2 = _sc_agg16(h2p, edges4)
    b2p = jnp.tile(b2.reshape(1, 16), (1, 8))
    gmask = jnp.kron(jnp.eye(8, dtype=jnp.float32),
                     jnp.ones((16, 16), jnp.float32))
    out = _tc_softmax(parts2, deg_dp, b2p, gmask)
    return out.reshape(R, 16)[:N]


# speedup vs baseline: 26.1329x; 1.0790x over previous
import functools

import jax
import jax.numpy as jnp
from jax import lax
from jax.experimental import pallas as pl
from jax.experimental.pallas import tpu as pltpu
from jax.experimental.pallas import tpu_sc as plsc

N = 10000
R = 10240
E = 320000
NC = 2
NS = 16
NW = NC * NS
CH = 100
KC = 100
RPW = R // NS
DCH = 80
DKC = RPW // DCH
RP8 = R // 8

_mesh = plsc.VectorSubcoreMesh(core_axis_name="c", subcore_axis_name="s")
_sc_params = pltpu.CompilerParams(use_tc_tiling_on_sc=False,
                                  needs_layout_passes=False)


def _fill_rows(ref, nrows, ncols, value):
    vec = jnp.full((16,), value, jnp.float32)

    def body(i, carry):
        for k in range(ncols // 16):
            ref[i, pl.ds(16 * k, 16)] = vec
        return carry

    lax.fori_loop(0, nrows, body, 0)


@functools.partial(
    pl.kernel,
    out_type=(
        jax.ShapeDtypeStruct((NC, R), jnp.float32),
        jax.ShapeDtypeStruct((NC, R), jnp.float32),
        jax.ShapeDtypeStruct((NC, RP8, 128), jnp.float32),
    ),
    mesh=_mesh,
    compiler_params=_sc_params,
    scratch_types=[
        pltpu.VMEM((KC, CH), jnp.int32),
        pltpu.VMEM((KC, CH), jnp.int32),
        pltpu.VMEM((CH, 16), jnp.float32),
        pltpu.VMEM((DCH, 16), jnp.float32),
        pltpu.VMEM((RPW,), jnp.float32),
        pltpu.VMEM((DCH * 16 // 128, 128), jnp.float32),
        pltpu.VMEM_SHARED((R, 16), jnp.float32),
        pltpu.VMEM_SHARED((R, 16), jnp.float32),
        pltpu.SemaphoreType.DMA,
        pltpu.SemaphoreType.DMA,
    ],
)
def _sc_degrees(edges_hbm, out_s_hbm, out_d_hbm, out_dp_hbm,
                src_v, dst_v, stage_v, ex_v, cvec, pview,
                acc_s, acc_d, sem_s, sem_d):
    cid = lax.axis_index("c")
    sid = lax.axis_index("s")
    wid = cid * NS + sid

    _fill_rows(stage_v, DCH, 16, 0.0)
    for k in range(DKC):
        r0 = sid * RPW + k * DCH
        pltpu.sync_copy(stage_v.at[pl.ds(0, DCH)], acc_s.at[pl.ds(r0, DCH)])
        pltpu.sync_copy(stage_v.at[pl.ds(0, DCH)], acc_d.at[pl.ds(r0, DCH)])
    _fill_rows(stage_v, CH, 16, 1.0)
    pltpu.sync_copy(edges_hbm.at[0, wid], src_v)
    pltpu.sync_copy(edges_hbm.at[1, wid], dst_v)
    plsc.subcore_barrier()

    descs = [None] * KC
    for j in range(KC):
        descs[j] = (
            pltpu.async_copy(stage_v.at[pl.ds(0, CH)], acc_s.at[src_v.at[j]],
                             sem_s, add=True),
            pltpu.async_copy(stage_v.at[pl.ds(0, CH)], acc_d.at[dst_v.at[j]],
                             sem_d, add=True),
        )
        if j >= 2:
            descs[j - 2][0].wait()
            descs[j - 2][1].wait()
    for j in range(max(KC - 2, 0), KC):
        descs[j][0].wait()
        descs[j][1].wait()
    plsc.subcore_barrier()

    iota = lax.iota(jnp.int32, 16)
    zcol = jnp.zeros((16,), jnp.int32)
    for acc, out_hbm, dp in ((acc_s, out_s_hbm, None), (acc_d, out_d_hbm, out_dp_hbm)):
        for k in range(DKC):
            r0 = sid * RPW + k * DCH
            pltpu.sync_copy(acc.at[pl.ds(r0, DCH)], ex_v)
            for m in range(DCH // 16):
                vals = plsc.load_gather(ex_v, [iota + 16 * m, zcol])
                cvec[pl.ds(k * DCH + 16 * m, 16)] = vals
            if dp is not None:
                for r in range(DCH):
                    pview[r // 8, pl.ds((r % 8) * 16, 16)] = ex_v[r, :]
                p0 = r0 * 16 // 128
                pltpu.sync_copy(pview, dp.at[cid, pl.ds(p0, DCH * 16 // 128)])
        pltpu.sync_copy(cvec, out_hbm.at[cid, pl.ds(sid * RPW, RPW)])


def _make_sc_agg(D, packed):
    out_shape = (NC, RP8, 128) if packed else (NC, R, D)
    pview_types = [pltpu.VMEM((DCH * D // 128, 128), jnp.float32)] if packed else []
    if packed:
        pview_types.append(pltpu.VMEM_SHARED((R, D), jnp.float32))

    @functools.partial(
        pl.kernel,
        out_type=jax.ShapeDtypeStruct(out_shape, jnp.float32),
        mesh=_mesh,
        compiler_params=_sc_params,
        scratch_types=[
            pltpu.VMEM((KC, CH), jnp.int32),
            pltpu.VMEM((KC, CH), jnp.int32),
            pltpu.VMEM((CH, D), jnp.float32),
            pltpu.VMEM((CH, D), jnp.float32),
            pltpu.VMEM_SHARED((R, D), jnp.float32),
            pltpu.SemaphoreType.DMA,
            pltpu.SemaphoreType.DMA,
        ] + pview_types,
    )
    def agg(table_hbm, edges_hbm, out_hbm,
            src_v, dst_v, buf0, buf1, acc, sem0, sem1, *maybe_pview):
        cid = lax.axis_index("c")
        sid = lax.axis_index("s")
        wid = cid * NS + sid
        bufs = (buf0, buf1)
        sems = (sem0, sem1)

        _fill_rows(buf0, DCH, D, 0.0)
        for k in range(DKC):
            r0 = sid * RPW + k * DCH
            pltpu.sync_copy(buf0.at[pl.ds(0, DCH)], acc.at[pl.ds(r0, DCH)])
        pltpu.sync_copy(edges_hbm.at[0, wid], src_v)
        pltpu.sync_copy(edges_hbm.at[1, wid], dst_v)
        if packed:
            table_sp = maybe_pview[1]
            for k in range(DKC):
                r0 = sid * RPW + k * DCH
                pltpu.sync_copy(table_hbm.at[pl.ds(r0, DCH)],
                                buf1.at[pl.ds(0, DCH)])
                pltpu.sync_copy(buf1.at[pl.ds(0, DCH)],
                                table_sp.at[pl.ds(r0, DCH)])
            table = table_sp
        else:
            table = table_hbm
        plsc.subcore_barrier()

        desc = [None] * KC
        desc[0] = pltpu.async_copy(table.at[src_v.at[0]], buf0, sem0)
        for j in range(KC):
            if j + 1 < KC:
                desc[j + 1] = pltpu.async_copy(
                    table.at[src_v.at[j + 1]], bufs[(j + 1) % 2],
                    sems[(j + 1) % 2])
            desc[j].wait()
            pltpu.sync_copy(bufs[j % 2], acc.at[dst_v.at[j]], add=True)
        plsc.subcore_barrier()

        for k in range(DKC):
            r0 = sid * RPW + k * DCH
            pltpu.sync_copy(acc.at[pl.ds(r0, DCH)], buf0.at[pl.ds(0, DCH)])
            if packed:
                pview = maybe_pview[0]
                for r in range(DCH):
                    pview[r // 8, pl.ds((r % 8) * 16, 16)] = buf0[r, :]
                p0 = (sid * RPW + k * DCH) * D // 128
                pltpu.sync_copy(pview, out_hbm.at[cid, pl.ds(p0, DCH * D // 128)])
            else:
                pltpu.sync_copy(buf0.at[pl.ds(0, DCH)],
                                out_hbm.at[cid, pl.ds(r0, DCH)])

    return agg


_sc_agg128 = _make_sc_agg(128, packed=False)
_sc_agg16 = _make_sc_agg(16, packed=True)


BLK = 1024
BLK2 = 512


def _norm_from(deg_ref, blk):
    deg = deg_ref[0, :] + deg_ref[1, :]
    norm = jnp.where(deg > 0, lax.rsqrt(jnp.maximum(deg, 1.0)), 0.0)
    return norm.reshape(blk, 1)


def _tc_h1p_body(x_ref, w_ref, degs_ref, o_ref):
    h = jnp.dot(x_ref[...], w_ref[...], preferred_element_type=jnp.float32)
    o_ref[...] = h * _norm_from(degs_ref, BLK)


def _tc_h1p(x_pad, W1, deg_s):
    return pl.pallas_call(
        _tc_h1p_body,
        grid=(R // BLK,),
        in_specs=[
            pl.BlockSpec((BLK, 128), lambda i: (i, 0)),
            pl.BlockSpec((128, 128), lambda i: (0, 0)),
            pl.BlockSpec((NC, BLK), lambda i: (0, i)),
        ],
        out_specs=pl.BlockSpec((BLK, 128), lambda i: (i, 0)),
        out_shape=jax.ShapeDtypeStruct((R, 128), jnp.float32),
    )(x_pad, W1, deg_s)


def _tc_mid_body(p_ref, degs_ref, degd_ref, b1_ref, w2_ref, o_ref):
    agg = p_ref[0] + p_ref[1]
    h = jax.nn.relu(agg * _norm_from(degd_ref, BLK) + b1_ref[...])
    h2 = jnp.dot(h, w2_ref[...], preferred_element_type=jnp.float32)
    o_ref[...] = h2 * _norm_from(degs_ref, BLK)


def _tc_mid(parts1, deg_s, deg_d, b1, W2):
    return pl.pallas_call(
        _tc_mid_body,
        grid=(R // BLK,),
        in_specs=[
            pl.BlockSpec((NC, BLK, 128), lambda i: (0, i, 0)),
            pl.BlockSpec((NC, BLK), lambda i: (0, i)),
            pl.BlockSpec((NC, BLK), lambda i: (0, i)),
            pl.BlockSpec((1, 128), lambda i: (0, 0)),
            pl.BlockSpec((128, 16), lambda i: (0, 0)),
        ],
        out_specs=pl.BlockSpec((BLK, 16), lambda i: (i, 0)),
        out_shape=jax.ShapeDtypeStruct((R, 16), jnp.float32),
    )(parts1, deg_s, deg_d, b1, W2)


PBLK = BLK2 * 16 // 128


def _tc_softmax_body(p_ref, degdp_ref, b2p_ref, gmask_ref, o_ref):
    agg = p_ref[0] + p_ref[1]
    deg = degdp_ref[0] + degdp_ref[1]
    norm = jnp.where(deg > 0, lax.rsqrt(jnp.maximum(deg, 1.0)), 0.0)
    z = agg * norm + b2p_ref[...]
    ez = jnp.exp(z - jnp.max(z, axis=1, keepdims=True))
    s = jnp.dot(ez, gmask_ref[...], preferred_element_type=jnp.float32,
                precision=lax.Precision.HIGHEST)
    o_ref[...] = ez / s


def _tc_softmax(parts2, deg_dp, b2p, gmask):
    return pl.pallas_call(
        _tc_softmax_body,
        grid=(R // BLK2,),
        in_specs=[
            pl.BlockSpec((NC, PBLK, 128), lambda i: (0, i, 0)),
            pl.BlockSpec((NC, PBLK, 128), lambda i: (0, i, 0)),
            pl.BlockSpec((1, 128), lambda i: (0, 0)),
            pl.BlockSpec((128, 128), lambda i: (0, 0)),
        ],
        out_specs=pl.BlockSpec((PBLK, 128), lambda i: (i, 0)),
        out_shape=jax.ShapeDtypeStruct((RP8, 128), jnp.float32),
    )(parts2, deg_dp, b2p, gmask)


def kernel(edge_index, inputs, W1, b1, W2, b2):
    edges4 = edge_index.astype(jnp.int32).reshape(2, NW, KC, CH)

    deg_s, deg_d, deg_dp = _sc_degrees(edges4)

    x_pad = jnp.pad(inputs, ((0, R - N), (0, 0)))
    h1p = _tc_h1p(x_pad, W1, deg_s)
    parts1 = _sc_agg128(h1p, edges4)
    h2p = _tc_mid(parts1, deg_s, deg_d, b1.reshape(1, 128), W2)
    parts2 = _sc_agg16(h2p, edges4)
    b2p = jnp.tile(b2.reshape(1, 16), (1, 8))
    gmask = jnp.kron(jnp.eye(8, dtype=jnp.float32),
                     jnp.ones((16, 16), jnp.float32))
    out = _tc_softmax(parts2, deg_dp, b2p, gmask)
    return out.reshape(R, 16)[:N]


# speedup vs baseline: 27.0500x; 1.0351x over previous
import functools

import jax
import jax.numpy as jnp
from jax import lax
from jax.experimental import pallas as pl
from jax.experimental.pallas import tpu as pltpu
from jax.experimental.pallas import tpu_sc as plsc

N = 10000
R = 10240
E = 320000
NC = 2
NS = 16
NW = NC * NS
CH = 100
KC = 100
RPW = R // NS
DCH = 80
DKC = RPW // DCH
RP8 = R // 8

_mesh = plsc.VectorSubcoreMesh(core_axis_name="c", subcore_axis_name="s")
_sc_params = pltpu.CompilerParams(use_tc_tiling_on_sc=False,
                                  needs_layout_passes=False)


def _fill_rows(ref, nrows, ncols, value):
    vec = jnp.full((16,), value, jnp.float32)

    def body(i, carry):
        for k in range(ncols // 16):
            ref[i, pl.ds(16 * k, 16)] = vec
        return carry

    lax.fori_loop(0, nrows, body, 0)


@functools.partial(
    pl.kernel,
    out_type=(
        jax.ShapeDtypeStruct((NC, R), jnp.float32),
        jax.ShapeDtypeStruct((NC, R), jnp.float32),
        jax.ShapeDtypeStruct((NC, RP8, 128), jnp.float32),
    ),
    mesh=_mesh,
    compiler_params=_sc_params,
    scratch_types=[
        pltpu.VMEM((KC, CH), jnp.int32),
        pltpu.VMEM((KC, CH), jnp.int32),
        pltpu.VMEM((CH, 16), jnp.float32),
        pltpu.VMEM((DCH, 16), jnp.float32),
        pltpu.VMEM((RPW,), jnp.float32),
        pltpu.VMEM((DCH * 16 // 128, 128), jnp.float32),
        pltpu.VMEM_SHARED((R, 16), jnp.float32),
        pltpu.VMEM_SHARED((R, 16), jnp.float32),
        pltpu.SemaphoreType.DMA,
        pltpu.SemaphoreType.DMA,
    ],
)
def _sc_degrees(edges_hbm, out_s_hbm, out_d_hbm, out_dp_hbm,
                src_v, dst_v, stage_v, ex_v, cvec, pview,
                acc_s, acc_d, sem_s, sem_d):
    cid = lax.axis_index("c")
    sid = lax.axis_index("s")
    wid = cid * NS + sid

    _fill_rows(stage_v, DCH, 16, 0.0)
    for k in range(DKC):
        r0 = sid * RPW + k * DCH
        pltpu.sync_copy(stage_v.at[pl.ds(0, DCH)], acc_s.at[pl.ds(r0, DCH)])
        pltpu.sync_copy(stage_v.at[pl.ds(0, DCH)], acc_d.at[pl.ds(r0, DCH)])
    _fill_rows(stage_v, CH, 16, 1.0)
    pltpu.sync_copy(edges_hbm.at[0, wid], src_v)
    pltpu.sync_copy(edges_hbm.at[1, wid], dst_v)
    plsc.subcore_barrier()

    descs = [None] * KC
    for j in range(KC):
        descs[j] = (
            pltpu.async_copy(stage_v.at[pl.ds(0, CH)], acc_s.at[src_v.at[j]],
                             sem_s, add=True),
            pltpu.async_copy(stage_v.at[pl.ds(0, CH)], acc_d.at[dst_v.at[j]],
                             sem_d, add=True),
        )
        if j >= 2:
            descs[j - 2][0].wait()
            descs[j - 2][1].wait()
    for j in range(max(KC - 2, 0), KC):
        descs[j][0].wait()
        descs[j][1].wait()
    plsc.subcore_barrier()

    iota = lax.iota(jnp.int32, 16)
    zcol = jnp.zeros((16,), jnp.int32)
    for acc, out_hbm, dp in ((acc_s, out_s_hbm, None), (acc_d, out_d_hbm, out_dp_hbm)):
        for k in range(DKC):
            r0 = sid * RPW + k * DCH
            pltpu.sync_copy(acc.at[pl.ds(r0, DCH)], ex_v)
            for m in range(DCH // 16):
                vals = plsc.load_gather(ex_v, [iota + 16 * m, zcol])
                cvec[pl.ds(k * DCH + 16 * m, 16)] = vals
            if dp is not None:
                for r in range(DCH):
                    pview[r // 8, pl.ds((r % 8) * 16, 16)] = ex_v[r, :]
                p0 = r0 * 16 // 128
                pltpu.sync_copy(pview, dp.at[cid, pl.ds(p0, DCH * 16 // 128)])
        pltpu.sync_copy(cvec, out_hbm.at[cid, pl.ds(sid * RPW, RPW)])


def _make_sc_agg(D, packed):
    out_shape = (NC, RP8, 128) if packed else (NC, R, D)
    pview_types = [pltpu.VMEM((DCH * D // 128, 128), jnp.float32)] if packed else []
    if packed:
        pview_types.append(pltpu.VMEM_SHARED((R, D), jnp.float32))

    @functools.partial(
        pl.kernel,
        out_type=jax.ShapeDtypeStruct(out_shape, jnp.float32),
        mesh=_mesh,
        compiler_params=_sc_params,
        scratch_types=[
            pltpu.VMEM((KC, CH), jnp.int32),
            pltpu.VMEM((KC, CH), jnp.int32),
            pltpu.VMEM((CH, D), jnp.float32),
            pltpu.VMEM((CH, D), jnp.float32),
            pltpu.VMEM_SHARED((R, D), jnp.float32),
            pltpu.SemaphoreType.DMA,
            pltpu.SemaphoreType.DMA,
        ] + pview_types,
    )
    def agg(table_hbm, edges_hbm, out_hbm,
            src_v, dst_v, buf0, buf1, acc, sem0, sem1, *maybe_pview):
        cid = lax.axis_index("c")
        sid = lax.axis_index("s")
        wid = cid * NS + sid
        bufs = (buf0, buf1)
        sems = (sem0, sem1)

        _fill_rows(buf0, DCH, D, 0.0)
        for k in range(DKC):
            r0 = sid * RPW + k * DCH
            pltpu.sync_copy(buf0.at[pl.ds(0, DCH)], acc.at[pl.ds(r0, DCH)])
        pltpu.sync_copy(edges_hbm.at[0, wid], src_v)
        pltpu.sync_copy(edges_hbm.at[1, wid], dst_v)
        if packed:
            table_sp = maybe_pview[1]
            for k in range(DKC):
                r0 = sid * RPW + k * DCH
                pltpu.sync_copy(table_hbm.at[pl.ds(r0, DCH)],
                                buf1.at[pl.ds(0, DCH)])
                pltpu.sync_copy(buf1.at[pl.ds(0, DCH)],
                                table_sp.at[pl.ds(r0, DCH)])
            table = table_sp
        else:
            table = table_hbm
        plsc.subcore_barrier()

        desc = [None] * KC
        desc[0] = pltpu.async_copy(table.at[src_v.at[0]], buf0, sem0)
        for j in range(KC):
            if j + 1 < KC:
                desc[j + 1] = pltpu.async_copy(
                    table.at[src_v.at[j + 1]], bufs[(j + 1) % 2],
                    sems[(j + 1) % 2])
            desc[j].wait()
            pltpu.sync_copy(bufs[j % 2], acc.at[dst_v.at[j]], add=True)
        plsc.subcore_barrier()

        for k in range(DKC):
            r0 = sid * RPW + k * DCH
            pltpu.sync_copy(acc.at[pl.ds(r0, DCH)], buf0.at[pl.ds(0, DCH)])
            if packed:
                pview = maybe_pview[0]
                for r in range(DCH):
                    pview[r // 8, pl.ds((r % 8) * 16, 16)] = buf0[r, :]
                p0 = (sid * RPW + k * DCH) * D // 128
                pltpu.sync_copy(pview, out_hbm.at[cid, pl.ds(p0, DCH * D // 128)])
            else:
                pltpu.sync_copy(buf0.at[pl.ds(0, DCH)],
                                out_hbm.at[cid, pl.ds(r0, DCH)])

    return agg


_sc_agg128 = _make_sc_agg(128, packed=False)
_sc_agg16 = _make_sc_agg(16, packed=True)


BLK = 1024
BLK2 = 2048


def _norm_from(deg_ref, blk):
    deg = deg_ref[0, :] + deg_ref[1, :]
    norm = jnp.where(deg > 0, lax.rsqrt(jnp.maximum(deg, 1.0)), 0.0)
    return norm.reshape(blk, 1)


def _tc_mm_body(x_ref, w_ref, o_ref):
    o_ref[...] = jnp.dot(x_ref[...], w_ref[...],
                         preferred_element_type=jnp.float32)


def _tc_mm(x_pad, W1):
    return pl.pallas_call(
        _tc_mm_body,
        grid=(R // BLK,),
        in_specs=[
            pl.BlockSpec((BLK, 128), lambda i: (i, 0)),
            pl.BlockSpec((128, 128), lambda i: (0, 0)),
        ],
        out_specs=pl.BlockSpec((BLK, 128), lambda i: (i, 0)),
        out_shape=jax.ShapeDtypeStruct((R, 128), jnp.float32),
    )(x_pad, W1)


def _tc_scale_body(m_ref, degs_ref, o_ref):
    o_ref[...] = m_ref[...] * _norm_from(degs_ref, BLK)


def _tc_scale(mm, deg_s):
    return pl.pallas_call(
        _tc_scale_body,
        grid=(R // BLK,),
        in_specs=[
            pl.BlockSpec((BLK, 128), lambda i: (i, 0)),
            pl.BlockSpec((NC, BLK), lambda i: (0, i)),
        ],
        out_specs=pl.BlockSpec((BLK, 128), lambda i: (i, 0)),
        out_shape=jax.ShapeDtypeStruct((R, 128), jnp.float32),
    )(mm, deg_s)


def _tc_mid_body(p_ref, degs_ref, degd_ref, b1_ref, w2_ref, o_ref):
    agg = p_ref[0] + p_ref[1]
    h = jax.nn.relu(agg * _norm_from(degd_ref, BLK) + b1_ref[...])
    h2 = jnp.dot(h, w2_ref[...], preferred_element_type=jnp.float32)
    o_ref[...] = h2 * _norm_from(degs_ref, BLK)


def _tc_mid(parts1, deg_s, deg_d, b1, W2):
    return pl.pallas_call(
        _tc_mid_body,
        grid=(R // BLK,),
        in_specs=[
            pl.BlockSpec((NC, BLK, 128), lambda i: (0, i, 0)),
            pl.BlockSpec((NC, BLK), lambda i: (0, i)),
            pl.BlockSpec((NC, BLK), lambda i: (0, i)),
            pl.BlockSpec((1, 128), lambda i: (0, 0)),
            pl.BlockSpec((128, 16), lambda i: (0, 0)),
        ],
        out_specs=pl.BlockSpec((BLK, 16), lambda i: (i, 0)),
        out_shape=jax.ShapeDtypeStruct((R, 16), jnp.float32),
    )(parts1, deg_s, deg_d, b1, W2)


PBLK = BLK2 * 16 // 128


def _tc_softmax_body(p_ref, degdp_ref, b2p_ref, gmask_ref, o_ref):
    agg = p_ref[0] + p_ref[1]
    deg = degdp_ref[0] + degdp_ref[1]
    norm = jnp.where(deg > 0, lax.rsqrt(jnp.maximum(deg, 1.0)), 0.0)
    z = agg * norm + b2p_ref[...]
    ez = jnp.exp(z - jnp.max(z, axis=1, keepdims=True))
    s = jnp.dot(ez, gmask_ref[...], preferred_element_type=jnp.float32,
                precision=lax.Precision.HIGHEST)
    o_ref[...] = ez / s


def _tc_softmax(parts2, deg_dp, b2p, gmask):
    return pl.pallas_call(
        _tc_softmax_body,
        grid=(R // BLK2,),
        in_specs=[
            pl.BlockSpec((NC, PBLK, 128), lambda i: (0, i, 0)),
            pl.BlockSpec((NC, PBLK, 128), lambda i: (0, i, 0)),
            pl.BlockSpec((1, 128), lambda i: (0, 0)),
            pl.BlockSpec((128, 128), lambda i: (0, 0)),
        ],
        out_specs=pl.BlockSpec((PBLK, 128), lambda i: (i, 0)),
        out_shape=jax.ShapeDtypeStruct((RP8, 128), jnp.float32),
    )(parts2, deg_dp, b2p, gmask)


def kernel(edge_index, inputs, W1, b1, W2, b2):
    edges4 = edge_index.astype(jnp.int32).reshape(2, NW, KC, CH)

    deg_s, deg_d, deg_dp = _sc_degrees(edges4)

    x_pad = jnp.pad(inputs, ((0, R - N), (0, 0)))
    h1p = _tc_scale(_tc_mm(x_pad, W1), deg_s)
    parts1 = _sc_agg128(h1p, edges4)
    h2p = _tc_mid(parts1, deg_s, deg_d, b1.reshape(1, 128), W2)
    parts2 = _sc_agg16(h2p, edges4)
    b2p = jnp.tile(b2.reshape(1, 16), (1, 8))
    gmask = jnp.kron(jnp.eye(8, dtype=jnp.float32),
                     jnp.ones((16, 16), jnp.float32))
    out = _tc_softmax(parts2, deg_dp, b2p, gmask)
    return out.reshape(R, 16)[:N]
